# Initial kernel scaffold; baseline (speedup 1.0000x reference)
import jax
import jax.numpy as jnp
from jax.experimental import pallas as pl


def kernel(x, edge_index, edge_attr, WQ, bQ, WK, bK, WE, bE, WV, bV, Aw, VeRow):
    raise NotImplementedError("write your pallas kernel here")


# speedup vs baseline: 18.2923x; 18.2923x over previous
import functools

import jax
import jax.numpy as jnp
import numpy as np
from jax.experimental import pallas as pl

N = 10000
E_EDGES = 320000
IN_DIM = 128
H_DIM = 16
HEADS = 8
HD = H_DIM * HEADS
CLAMP = 5.0

_NB = 1000
_EB = 2000


def _proj_body(x_ref, wk, bk, wq, bq, wv, bv, k_out, q_out, v_out):
    xb = x_ref[...]
    k_out[...] = jnp.dot(xb, wk[...], preferred_element_type=jnp.float32) + bk[...]
    q_out[...] = jnp.dot(xb, wq[...], preferred_element_type=jnp.float32) + bq[...]
    v_out[...] = jnp.dot(xb, wv[...], preferred_element_type=jnp.float32) + bv[...]


def _proj(x, WK, bK, WQ, bQ, WV, bV):
    full = lambda s: pl.BlockSpec(s, lambda i: (0,) * len(s))
    nspec = pl.BlockSpec((_NB, HD), lambda i: (i, 0))
    return pl.pallas_call(
        _proj_body,
        grid=(N // _NB,),
        in_specs=[pl.BlockSpec((_NB, IN_DIM), lambda i: (i, 0)),
                  full((IN_DIM, HD)), full((1, HD)),
                  full((IN_DIM, HD)), full((1, HD)),
                  full((IN_DIM, HD)), full((1, HD))],
        out_specs=[nspec, nspec, nspec],
        out_shape=[jax.ShapeDtypeStruct((N, HD), jnp.float32)] * 3,
    )(x, WK, bK.reshape(1, HD), WQ, bQ.reshape(1, HD), WV, bV.reshape(1, HD))


def _edge_body(ea_ref, g_ref, we1, be1, we2, be2, awb, r8,
               eout_ref, p_ref, m2_ref):
    ea = ea_ref[...]
    ex1 = jnp.dot(ea, we1[...], preferred_element_type=jnp.float32) + be1[...]
    ex2 = jnp.dot(ea, we2[...], preferred_element_type=jnp.float32) + be2[...]
    s2 = ex1 * ex2
    score2 = jnp.sqrt(jax.nn.relu(s2)) - jnp.sqrt(jax.nn.relu(-s2))
    conn = g_ref[...] + score2
    eout_ref[...] = conn
    score = jnp.dot(conn, awb[...], preferred_element_type=jnp.float32)
    p = jnp.exp(jnp.clip(score, -CLAMP, CLAMP))
    p_ref[...] = p
    prep = jnp.dot(p, r8[...], preferred_element_type=jnp.float32)
    m2_ref[...] = prep * conn


def _edge_stage(edge_attr, G, WE1, bE1, WE2, bE2, AwBlock, R8):
    full = lambda s: pl.BlockSpec(s, lambda i: (0,) * len(s))
    espec = pl.BlockSpec((_EB, HD), lambda i: (i, 0))
    return pl.pallas_call(
        _edge_body,
        grid=(E_EDGES // _EB,),
        in_specs=[espec, espec,
                  full((IN_DIM, HD)), full((1, HD)),
                  full((IN_DIM, HD)), full((1, HD)),
                  full((IN_DIM, HEADS)), full((HEADS, HD))],
        out_specs=[espec, pl.BlockSpec((_EB, HEADS), lambda i: (i, 0)), espec],
        out_shape=[jax.ShapeDtypeStruct((E_EDGES, HD), jnp.float32),
                   jax.ShapeDtypeStruct((E_EDGES, HEADS), jnp.float32),
                   jax.ShapeDtypeStruct((E_EDGES, HD), jnp.float32)],
    )(edge_attr, G, WE1, bE1.reshape(1, HD), WE2, bE2.reshape(1, HD),
      AwBlock, R8)


def _node_body(q_ref, s_ref, av_ref, ac_ref, r8, veb, out_ref):
    inv = 1.0 / (s_ref[...] + 1e-30)
    inv_rep = jnp.dot(inv, r8[...], preferred_element_type=jnp.float32)
    rv = ac_ref[...] * inv_rep
    out_ref[...] = (q_ref[...] + av_ref[...] * inv_rep
                    + jnp.dot(rv, veb[...], preferred_element_type=jnp.float32))


def _node_stage(Q, S, accV, accC, R8, VeBlock):
    full = lambda s: pl.BlockSpec(s, lambda i: (0,) * len(s))
    nspec = pl.BlockSpec((_NB, HD), lambda i: (i, 0))
    return pl.pallas_call(
        _node_body,
        grid=(N // _NB,),
        in_specs=[nspec, pl.BlockSpec((_NB, HEADS), lambda i: (i, 0)),
                  nspec, nspec, full((HEADS, HD)), full((HD, HD))],
        out_specs=nspec,
        out_shape=jax.ShapeDtypeStruct((N, HD), jnp.float32),
    )(Q, S, accV, accC, R8, VeBlock)


def kernel(x, edge_index, edge_attr, WQ, bQ, WK, bK, WE, bE, WV, bV, Aw, VeRow):
    src = edge_index[0]
    dst = edge_index[1]

    h = np.arange(HEADS)
    j = np.arange(H_DIM)
    perm1 = (32 * h[:, None] + j[None, :]).reshape(-1)
    perm2 = (32 * h[:, None] + 16 + j[None, :]).reshape(-1)
    WE1, bE1 = WE[:, perm1], bE[perm1]
    WE2, bE2 = WE[:, perm2], bE[perm2]

    rows = jnp.arange(HD)
    hcol = jnp.repeat(jnp.arange(HEADS), H_DIM)
    AwBlock = jnp.zeros((HD, HEADS), jnp.float32).at[rows, hcol].set(
        Aw[:, :, 0].T.reshape(HD))
    R8 = jnp.zeros((HEADS, HD), jnp.float32).at[hcol, rows].set(1.0)

    h_i = jnp.repeat(jnp.arange(HEADS), H_DIM * H_DIM)
    d_i = jnp.tile(jnp.repeat(jnp.arange(H_DIM), H_DIM), HEADS)
    c_i = jnp.tile(jnp.arange(H_DIM), HEADS * H_DIM)
    VeBlock = jnp.zeros((HD, HD), jnp.float32).at[
        16 * h_i + d_i, 16 * h_i + c_i].set(VeRow[d_i, h_i, c_i])

    K, Q, V = _proj(x, WK, bK, WQ, bQ, WV, bV)

    G = K[src] + Q[dst]

    e_out, p, m2 = _edge_stage(edge_attr, G, WE1, bE1, WE2, bE2, AwBlock, R8)

    S = jax.ops.segment_sum(p, dst, num_segments=N)
    prep = jnp.repeat(p, H_DIM, axis=1)
    accV = jax.ops.segment_sum(prep * V[src], dst, num_segments=N)
    accC = jax.ops.segment_sum(m2, dst, num_segments=N)

    n_out = _node_stage(Q, S, accV, accC, R8, VeBlock)
    return (n_out, e_out)


# speedup vs baseline: 29.8829x; 1.6336x over previous
import functools

import jax
import jax.numpy as jnp
import numpy as np
from jax import lax
from jax.experimental import pallas as pl
from jax.experimental.pallas import tpu as pltpu
from jax.experimental.pallas import tpu_sc as plsc

N = 10000
E_EDGES = 320000
IN_DIM = 128
H_DIM = 16
HEADS = 8
HD = H_DIM * HEADS
CLAMP = 5.0

_NB = 1000
_EB = 2000


def _proj_body(x_ref, wk, bk, wq, bq, wv, bv, k_out, q_out, v_out):
    xb = x_ref[...]
    k_out[...] = jnp.dot(xb, wk[...], preferred_element_type=jnp.float32) + bk[...]
    q_out[...] = jnp.dot(xb, wq[...], preferred_element_type=jnp.float32) + bq[...]
    v_out[...] = jnp.dot(xb, wv[...], preferred_element_type=jnp.float32) + bv[...]


def _proj(x, WK, bK, WQ, bQ, WV, bV):
    full = lambda s: pl.BlockSpec(s, lambda i: (0,) * len(s))
    nspec = pl.BlockSpec((_NB, HD), lambda i: (i, 0))
    return pl.pallas_call(
        _proj_body,
        grid=(N // _NB,),
        in_specs=[pl.BlockSpec((_NB, IN_DIM), lambda i: (i, 0)),
                  full((IN_DIM, HD)), full((1, HD)),
                  full((IN_DIM, HD)), full((1, HD)),
                  full((IN_DIM, HD)), full((1, HD))],
        out_specs=[nspec, nspec, nspec],
        out_shape=[jax.ShapeDtypeStruct((N, HD), jnp.float32)] * 3,
    )(x, WK, bK.reshape(1, HD), WQ, bQ.reshape(1, HD), WV, bV.reshape(1, HD))


def _edge_body(ea_ref, g_ref, we1, be1, we2, be2, awb, r8,
               eout_ref, p_ref, m2_ref):
    ea = ea_ref[...]
    ex1 = jnp.dot(ea, we1[...], preferred_element_type=jnp.float32) + be1[...]
    ex2 = jnp.dot(ea, we2[...], preferred_element_type=jnp.float32) + be2[...]
    s2 = ex1 * ex2
    score2 = jnp.sqrt(jax.nn.relu(s2)) - jnp.sqrt(jax.nn.relu(-s2))
    conn = g_ref[...] + score2
    eout_ref[...] = conn
    score = jnp.dot(conn, awb[...], preferred_element_type=jnp.float32)
    p = jnp.exp(jnp.clip(score, -CLAMP, CLAMP))
    p_ref[...] = p
    prep = jnp.dot(p, r8[...], preferred_element_type=jnp.float32)
    m2_ref[...] = prep * conn


def _edge_stage(edge_attr, G, WE1, bE1, WE2, bE2, AwBlock, R8):
    full = lambda s: pl.BlockSpec(s, lambda i: (0,) * len(s))
    espec = pl.BlockSpec((_EB, HD), lambda i: (i, 0))
    return pl.pallas_call(
        _edge_body,
        grid=(E_EDGES // _EB,),
        in_specs=[espec, espec,
                  full((IN_DIM, HD)), full((1, HD)),
                  full((IN_DIM, HD)), full((1, HD)),
                  full((IN_DIM, HEADS)), full((HEADS, HD))],
        out_specs=[espec, pl.BlockSpec((_EB, HEADS), lambda i: (i, 0)), espec],
        out_shape=[jax.ShapeDtypeStruct((E_EDGES, HD), jnp.float32),
                   jax.ShapeDtypeStruct((E_EDGES, HEADS), jnp.float32),
                   jax.ShapeDtypeStruct((E_EDGES, HD), jnp.float32)],
    )(edge_attr, G, WE1, bE1.reshape(1, HD), WE2, bE2.reshape(1, HD),
      AwBlock, R8)


_C = 128
_NCHUNK = E_EDGES // _C
_NSUB = 16
_NPAD = 10240
_NROW = _NPAD // _NSUB
_SROW = _NPAD // 16
_GDN = lax.GatherDimensionNumbers(
    offset_dims=(), collapsed_slice_dims=(0,), start_index_map=(0,))


def _bcast16(vec, idxvec):
    return lax.gather(vec, idxvec, _GDN, (1,),
                      mode=lax.GatherScatterMode.PROMISE_IN_BOUNDS)


def _p4_body(dstE, srcE, pflat, m2, vN, zin, out0, out1,
             sh, idxb, sidx, pbuf, vbuf, pay, sem):
    cid = lax.axis_index("c")
    sid = lax.axis_index("s")
    iota16 = lax.iota(jnp.int32, 16)
    splat = [(iota16 * 0 + k).reshape(16, 1) for k in range(16)]
    row0 = sid * _NROW
    pltpu.sync_copy(zin.at[pl.ds(row0, _NROW), :], sh.at[pl.ds(row0, _NROW), :])
    plsc.subcore_barrier()

    rem = _NCHUNK - (_NCHUNK // _NSUB) * _NSUB
    nloc = (_NCHUNK // _NSUB) + jnp.where(sid < rem, 1, 0)

    @pl.when(cid == 0)
    def _():
        def body0(i, carry):
            base = (i * _NSUB + sid) * _C
            pltpu.sync_copy(dstE.at[pl.ds(base, _C)], idxb.at[0])
            pltpu.sync_copy(srcE.at[pl.ds(base, _C)], sidx)
            pltpu.sync_copy(pflat.at[pl.ds(base * HEADS, _C * HEADS)], pbuf)
            pltpu.async_copy(vN.at[sidx], vbuf, sem).wait()
            for q2 in range(_C // 2):
                pp = pbuf[pl.ds(q2 * 16, 16)]
                for a in range(2):
                    r = 2 * q2 + a
                    for h in range(HEADS):
                        w = _bcast16(pp, splat[a * HEADS + h])
                        pay[r, pl.ds(h * H_DIM, 16)] = (
                            vbuf[r, pl.ds(h * H_DIM, 16)] * w)
            pltpu.sync_copy(pay, sh.at[idxb.at[0]], add=True)
            return carry
        lax.fori_loop(0, nloc, body0, 0)

    @pl.when(cid == 1)
    def _():
        def body1(i, carry):
            base = (i * _NSUB + sid) * _C
            pltpu.sync_copy(dstE.at[pl.ds(base, _C)], idxb.at[0])
            pltpu.sync_copy(m2.at[pl.ds(base, _C), :], pay)
            pltpu.sync_copy(pay, sh.at[idxb.at[0]], add=True)
            return carry
        lax.fori_loop(0, nloc, body1, 0)

    plsc.subcore_barrier()

    @pl.when(cid == 0)
    def _():
        pltpu.sync_copy(sh.at[pl.ds(row0, _NROW), :],
                        out0.at[pl.ds(row0, _NROW), :])

    @pl.when(cid == 1)
    def _():
        pltpu.sync_copy(sh.at[pl.ds(row0, _NROW), :],
                        out1.at[pl.ds(row0, _NROW), :])


def _scatter_stage(dst, src, pflat, m2, V, zin):
    mesh = plsc.VectorSubcoreMesh(core_axis_name="c", subcore_axis_name="s")
    f = pl.kernel(
        _p4_body,
        out_type=[jax.ShapeDtypeStruct((_NPAD, HD), jnp.float32),
                  jax.ShapeDtypeStruct((_NPAD, HD), jnp.float32)],
        mesh=mesh,
        compiler_params=pltpu.CompilerParams(needs_layout_passes=False),
        scratch_types=[
            pltpu.VMEM_SHARED((_NPAD, HD), jnp.float32),
            pltpu.VMEM((1, _C), jnp.int32),
            pltpu.VMEM((_C,), jnp.int32),
            pltpu.VMEM((_C * HEADS,), jnp.float32),
            pltpu.VMEM((_C, HD), jnp.float32),
            pltpu.VMEM((_C, HD), jnp.float32),
            pltpu.SemaphoreType.DMA,
        ],
    )
    return f(dst, src, pflat, m2, V, zin)


def _p4b_body(dstE, pflat, outS0, outS1, sloc, idxb, pbuf):
    cid = lax.axis_index("c")
    sid = lax.axis_index("s")
    wid = sid * 2 + cid
    iota16 = lax.iota(jnp.int32, 16)
    splat = [(iota16 * 0 + k).reshape(16, 1) for k in range(16)]
    msk8 = iota16 < 8
    ioff = lax.bitwise_and(iota16, 7)
    zf = iota16.astype(jnp.float32) * 0.0

    def zbody(i, carry):
        for j in range(8):
            sloc[pl.ds(i * 128 + j * 16, 16)] = zf
        return carry
    lax.fori_loop(0, _SROW * HD // 128, zbody, 0)

    nw = _NCHUNK // 32
    rem = _NCHUNK - nw * 32
    nloc = nw + jnp.where(wid < rem, 1, 0)

    def body(i, carry):
        base = (i * 32 + wid) * _C
        pltpu.sync_copy(dstE.at[pl.ds(base, _C)], idxb.at[0])
        pltpu.sync_copy(pflat.at[pl.ds(base * HEADS, _C * HEADS)], pbuf)
        for q2 in range(_C // 2):
            pp = pbuf[pl.ds(q2 * 16, 16)]
            if q2 % 8 == 0:
                dwin = idxb[0, pl.ds(q2 * 2, 16)]
            d0 = _bcast16(dwin, splat[(2 * q2) % 16])
            d1 = _bcast16(dwin, splat[(2 * q2 + 1) % 16])
            plsc.addupdate_scatter(sloc, [d0 * HEADS + ioff], pp, mask=msk8)
            plsc.addupdate_scatter(sloc, [d1 * HEADS + ioff], pp, mask=~msk8)
        return carry
    lax.fori_loop(0, nloc, body, 0)

    @pl.when(cid == 0)
    def _():
        pltpu.sync_copy(sloc, outS0.at[sid])

    @pl.when(cid == 1)
    def _():
        pltpu.sync_copy(sloc, outS1.at[sid])


def _s_stage(dst, pflat):
    mesh = plsc.VectorSubcoreMesh(core_axis_name="c", subcore_axis_name="s")
    f = pl.kernel(
        _p4b_body,
        out_type=[jax.ShapeDtypeStruct((_NSUB, _SROW * HD), jnp.float32)] * 2,
        mesh=mesh,
        compiler_params=pltpu.CompilerParams(needs_layout_passes=False),
        scratch_types=[
            pltpu.VMEM((_SROW * HD,), jnp.float32),
            pltpu.VMEM((1, _C), jnp.int32),
            pltpu.VMEM((_C * HEADS,), jnp.float32),
        ],
    )
    return f(dst, pflat)


def _sum32_body(a_ref, b_ref, out_ref):
    out_ref[...] = jnp.sum(a_ref[...], axis=0) + jnp.sum(b_ref[...], axis=0)


def _sum32(t0, t1):
    blk = _SROW * HD // 8
    return pl.pallas_call(
        _sum32_body,
        grid=(8,),
        in_specs=[pl.BlockSpec((_NSUB, blk), lambda i: (0, i))] * 2,
        out_specs=pl.BlockSpec((blk,), lambda i: (i,)),
        out_shape=jax.ShapeDtypeStruct((_SROW * HD,), jnp.float32),
    )(t0, t1)


def _node_body(q_ref, s_ref, av_ref, ac_ref, r8, veb, out_ref):
    inv = 1.0 / (s_ref[...] + 1e-30)
    inv_rep = jnp.dot(inv, r8[...], preferred_element_type=jnp.float32)
    rv = ac_ref[...] * inv_rep
    out_ref[...] = (q_ref[...] + av_ref[...] * inv_rep
                    + jnp.dot(rv, veb[...], preferred_element_type=jnp.float32))


def _node_stage(Q, S, accV, accC, R8, VeBlock):
    full = lambda s: pl.BlockSpec(s, lambda i: (0,) * len(s))
    nspec = pl.BlockSpec((_NB, HD), lambda i: (i, 0))
    return pl.pallas_call(
        _node_body,
        grid=(N // _NB,),
        in_specs=[nspec, pl.BlockSpec((_NB, HEADS), lambda i: (i, 0)),
                  nspec, nspec, full((HEADS, HD)), full((HD, HD))],
        out_specs=nspec,
        out_shape=jax.ShapeDtypeStruct((N, HD), jnp.float32),
    )(Q, S, accV, accC, R8, VeBlock)


def kernel(x, edge_index, edge_attr, WQ, bQ, WK, bK, WE, bE, WV, bV, Aw, VeRow):
    src = edge_index[0]
    dst = edge_index[1]

    h = np.arange(HEADS)
    j = np.arange(H_DIM)
    perm1 = (32 * h[:, None] + j[None, :]).reshape(-1)
    perm2 = (32 * h[:, None] + 16 + j[None, :]).reshape(-1)
    WE1, bE1 = WE[:, perm1], bE[perm1]
    WE2, bE2 = WE[:, perm2], bE[perm2]

    rows = jnp.arange(HD)
    hcol = jnp.repeat(jnp.arange(HEADS), H_DIM)
    AwBlock = jnp.zeros((HD, HEADS), jnp.float32).at[rows, hcol].set(
        Aw[:, :, 0].T.reshape(HD))
    R8 = jnp.zeros((HEADS, HD), jnp.float32).at[hcol, rows].set(1.0)

    h_i = jnp.repeat(jnp.arange(HEADS), H_DIM * H_DIM)
    d_i = jnp.tile(jnp.repeat(jnp.arange(H_DIM), H_DIM), HEADS)
    c_i = jnp.tile(jnp.arange(H_DIM), HEADS * H_DIM)
    VeBlock = jnp.zeros((HD, HD), jnp.float32).at[
        16 * h_i + d_i, 16 * h_i + c_i].set(VeRow[d_i, h_i, c_i])

    K, Q, V = _proj(x, WK, bK, WQ, bQ, WV, bV)

    G = K[src] + Q[dst]

    e_out, p, m2 = _edge_stage(edge_attr, G, WE1, bE1, WE2, bE2, AwBlock, R8)

    zin = jnp.zeros((_NPAD, HD), jnp.float32)
    pflat = p.reshape(-1)
    out0, out1 = _scatter_stage(dst, src, pflat, m2, V, zin)
    outS0, outS1 = _s_stage(dst, pflat)
    accV = out0[:N]
    accC = out1[:N]
    S = _sum32(outS0, outS1).reshape(_NPAD, HEADS)[:N]

    n_out = _node_stage(Q, S, accV, accC, R8, VeBlock)
    return (n_out, e_out)


# speedup vs baseline: 40.3565x; 1.3505x over previous
import functools

import jax
import jax.numpy as jnp
import numpy as np
from jax import lax
from jax.experimental import pallas as pl
from jax.experimental.pallas import tpu as pltpu
from jax.experimental.pallas import tpu_sc as plsc

N = 10000
E_EDGES = 320000
IN_DIM = 128
H_DIM = 16
HEADS = 8
HD = H_DIM * HEADS
CLAMP = 5.0

_NB = 1000
_EB = 2000


def _proj_body(x_ref, wk, bk, wq, bq, wv, bv, k_out, q_out, v_out):
    xb = x_ref[...]
    k_out[...] = jnp.dot(xb, wk[...], preferred_element_type=jnp.float32) + bk[...]
    q_out[...] = jnp.dot(xb, wq[...], preferred_element_type=jnp.float32) + bq[...]
    v_out[...] = jnp.dot(xb, wv[...], preferred_element_type=jnp.float32) + bv[...]


def _proj(x, WK, bK, WQ, bQ, WV, bV):
    full = lambda s: pl.BlockSpec(s, lambda i: (0,) * len(s))
    nspec = pl.BlockSpec((_NB, HD), lambda i: (i, 0))
    return pl.pallas_call(
        _proj_body,
        grid=(N // _NB,),
        in_specs=[pl.BlockSpec((_NB, IN_DIM), lambda i: (i, 0)),
                  full((IN_DIM, HD)), full((1, HD)),
                  full((IN_DIM, HD)), full((1, HD)),
                  full((IN_DIM, HD)), full((1, HD))],
        out_specs=[nspec, nspec, nspec],
        out_shape=[jax.ShapeDtypeStruct((N, HD), jnp.float32)] * 3,
    )(x, WK, bK.reshape(1, HD), WQ, bQ.reshape(1, HD), WV, bV.reshape(1, HD))


def _edge_body(ea_ref, g_ref, we1, be1, we2, be2, awb, r8,
               eout_ref, p_ref, m2_ref):
    ea = ea_ref[...]
    ex1 = jnp.dot(ea, we1[...], preferred_element_type=jnp.float32) + be1[...]
    ex2 = jnp.dot(ea, we2[...], preferred_element_type=jnp.float32) + be2[...]
    s2 = ex1 * ex2
    score2 = jnp.sqrt(jax.nn.relu(s2)) - jnp.sqrt(jax.nn.relu(-s2))
    conn = g_ref[...] + score2
    eout_ref[...] = conn
    score = jnp.dot(conn, awb[...], preferred_element_type=jnp.float32)
    p = jnp.exp(jnp.clip(score, -CLAMP, CLAMP))
    p_ref[...] = p
    prep = jnp.dot(p, r8[...], preferred_element_type=jnp.float32)
    m2_ref[...] = prep * conn


def _edge_stage(edge_attr, G, WE1, bE1, WE2, bE2, AwBlock, R8):
    full = lambda s: pl.BlockSpec(s, lambda i: (0,) * len(s))
    espec = pl.BlockSpec((_EB, HD), lambda i: (i, 0))
    return pl.pallas_call(
        _edge_body,
        grid=(E_EDGES // _EB,),
        in_specs=[espec, espec,
                  full((IN_DIM, HD)), full((1, HD)),
                  full((IN_DIM, HD)), full((1, HD)),
                  full((IN_DIM, HEADS)), full((HEADS, HD))],
        out_specs=[espec, pl.BlockSpec((_EB, HEADS), lambda i: (i, 0)), espec],
        out_shape=[jax.ShapeDtypeStruct((E_EDGES, HD), jnp.float32),
                   jax.ShapeDtypeStruct((E_EDGES, HEADS), jnp.float32),
                   jax.ShapeDtypeStruct((E_EDGES, HD), jnp.float32)],
    )(edge_attr, G, WE1, bE1.reshape(1, HD), WE2, bE2.reshape(1, HD),
      AwBlock, R8)


_C = 128
_NCHUNK = E_EDGES // _C
_NSUB = 16
_NPAD = 10240
_NROW = _NPAD // _NSUB
_SROW = _NPAD // 16
_GDN = lax.GatherDimensionNumbers(
    offset_dims=(), collapsed_slice_dims=(0,), start_index_map=(0,))


def _bcast16(vec, idxvec):
    return lax.gather(vec, idxvec, _GDN, (1,),
                      mode=lax.GatherScatterMode.PROMISE_IN_BOUNDS)


def _p4_body(dstE, srcE, pflat, m2, vN, zin, out0, out1,
             sh, idxb, sidx, pbuf, vbuf, pay, sem):
    cid = lax.axis_index("c")
    sid = lax.axis_index("s")
    iota16 = lax.iota(jnp.int32, 16)
    splat = [(iota16 * 0 + k).reshape(16, 1) for k in range(16)]
    row0 = sid * _NROW
    pltpu.sync_copy(zin.at[pl.ds(row0, _NROW), :], sh.at[pl.ds(row0, _NROW), :])
    plsc.subcore_barrier()

    rem = _NCHUNK - (_NCHUNK // _NSUB) * _NSUB
    nloc = (_NCHUNK // _NSUB) + jnp.where(sid < rem, 1, 0)

    @pl.when(cid == 0)
    def _():
        def body0(i, carry):
            base = (i * _NSUB + sid) * _C
            pltpu.sync_copy(dstE.at[pl.ds(base, _C)], idxb.at[0])
            pltpu.sync_copy(srcE.at[pl.ds(base, _C)], sidx)
            pltpu.sync_copy(pflat.at[pl.ds(base * HEADS, _C * HEADS)], pbuf)
            pltpu.async_copy(vN.at[sidx], vbuf, sem).wait()
            for q2 in range(_C // 2):
                pp = pbuf[pl.ds(q2 * 16, 16)]
                for a in range(2):
                    r = 2 * q2 + a
                    for h in range(HEADS):
                        w = _bcast16(pp, splat[a * HEADS + h])
                        pay[r, pl.ds(h * H_DIM, 16)] = (
                            vbuf[r, pl.ds(h * H_DIM, 16)] * w)
            pltpu.sync_copy(pay, sh.at[idxb.at[0]], add=True)
            return carry
        lax.fori_loop(0, nloc, body0, 0)

    @pl.when(cid == 1)
    def _():
        def body1(i, carry):
            base = (i * _NSUB + sid) * _C
            pltpu.sync_copy(dstE.at[pl.ds(base, _C)], idxb.at[0])
            pltpu.sync_copy(m2.at[pl.ds(base, _C), :], pay)
            pltpu.sync_copy(pay, sh.at[idxb.at[0]], add=True)
            return carry
        lax.fori_loop(0, nloc, body1, 0)

    plsc.subcore_barrier()

    @pl.when(cid == 0)
    def _():
        pltpu.sync_copy(sh.at[pl.ds(row0, _NROW), :],
                        out0.at[pl.ds(row0, _NROW), :])

    @pl.when(cid == 1)
    def _():
        pltpu.sync_copy(sh.at[pl.ds(row0, _NROW), :],
                        out1.at[pl.ds(row0, _NROW), :])


def _scatter_stage(dst, src, pflat, m2, V, zin):
    mesh = plsc.VectorSubcoreMesh(core_axis_name="c", subcore_axis_name="s")
    f = pl.kernel(
        _p4_body,
        out_type=[jax.ShapeDtypeStruct((_NPAD, HD), jnp.float32),
                  jax.ShapeDtypeStruct((_NPAD, HD), jnp.float32)],
        mesh=mesh,
        compiler_params=pltpu.CompilerParams(needs_layout_passes=False),
        scratch_types=[
            pltpu.VMEM_SHARED((_NPAD, HD), jnp.float32),
            pltpu.VMEM((1, _C), jnp.int32),
            pltpu.VMEM((_C,), jnp.int32),
            pltpu.VMEM((_C * HEADS,), jnp.float32),
            pltpu.VMEM((_C, HD), jnp.float32),
            pltpu.VMEM((_C, HD), jnp.float32),
            pltpu.SemaphoreType.DMA,
        ],
    )
    return f(dst, src, pflat, m2, V, zin)


def _p2_body(srcE, dstE, kN, qN, gE, idxs, idxd, bufk, bufq, sem):
    cid = lax.axis_index("c")
    sid = lax.axis_index("s")
    wid = sid * 2 + cid
    nw = _NCHUNK // 32
    rem = _NCHUNK - nw * 32
    nloc = nw + jnp.where(wid < rem, 1, 0)

    def body(i, carry):
        base = (i * 32 + wid) * _C
        pltpu.sync_copy(srcE.at[pl.ds(base, _C)], idxs)
        pltpu.sync_copy(dstE.at[pl.ds(base, _C)], idxd)
        ck = pltpu.async_copy(kN.at[idxs], bufk, sem)
        cq = pltpu.async_copy(qN.at[idxd], bufq, sem)
        ck.wait()
        cq.wait()
        for r in range(_C):
            for h in range(HEADS):
                sl = pl.ds(h * H_DIM, 16)
                bufk[r, sl] = bufk[r, sl] + bufq[r, sl]
        pltpu.sync_copy(bufk, gE.at[pl.ds(base, _C), :])
        return carry
    lax.fori_loop(0, nloc, body, 0)


def _gather_stage(src, dst, K, Q):
    mesh = plsc.VectorSubcoreMesh(core_axis_name="c", subcore_axis_name="s")
    f = pl.kernel(
        _p2_body,
        out_type=[jax.ShapeDtypeStruct((E_EDGES, HD), jnp.float32)],
        mesh=mesh,
        compiler_params=pltpu.CompilerParams(needs_layout_passes=False),
        scratch_types=[
            pltpu.VMEM((_C,), jnp.int32),
            pltpu.VMEM((_C,), jnp.int32),
            pltpu.VMEM((_C, HD), jnp.float32),
            pltpu.VMEM((_C, HD), jnp.float32),
            pltpu.SemaphoreType.DMA,
        ],
    )
    return f(src, dst, K, Q)[0]


def _p4b_body(dstE, pflat, outS0, outS1, sloc, idxb, pbuf):
    cid = lax.axis_index("c")
    sid = lax.axis_index("s")
    wid = sid * 2 + cid
    iota16 = lax.iota(jnp.int32, 16)
    splat = [(iota16 * 0 + k).reshape(16, 1) for k in range(16)]
    msk8 = iota16 < 8
    ioff = lax.bitwise_and(iota16, 7)
    zf = iota16.astype(jnp.float32) * 0.0

    def zbody(i, carry):
        for j in range(8):
            sloc[pl.ds(i * 128 + j * 16, 16)] = zf
        return carry
    lax.fori_loop(0, _SROW * HD // 128, zbody, 0)

    nw = _NCHUNK // 32
    rem = _NCHUNK - nw * 32
    nloc = nw + jnp.where(wid < rem, 1, 0)

    def body(i, carry):
        base = (i * 32 + wid) * _C
        pltpu.sync_copy(dstE.at[pl.ds(base, _C)], idxb.at[0])
        pltpu.sync_copy(pflat.at[pl.ds(base * HEADS, _C * HEADS)], pbuf)
        for q2 in range(_C // 2):
            pp = pbuf[pl.ds(q2 * 16, 16)]
            if q2 % 8 == 0:
                dwin = idxb[0, pl.ds(q2 * 2, 16)]
            d0 = _bcast16(dwin, splat[(2 * q2) % 16])
            d1 = _bcast16(dwin, splat[(2 * q2 + 1) % 16])
            plsc.addupdate_scatter(sloc, [d0 * HEADS + ioff], pp, mask=msk8)
            plsc.addupdate_scatter(sloc, [d1 * HEADS + ioff], pp, mask=~msk8)
        return carry
    lax.fori_loop(0, nloc, body, 0)

    @pl.when(cid == 0)
    def _():
        pltpu.sync_copy(sloc, outS0.at[sid])

    @pl.when(cid == 1)
    def _():
        pltpu.sync_copy(sloc, outS1.at[sid])


def _s_stage(dst, pflat):
    mesh = plsc.VectorSubcoreMesh(core_axis_name="c", subcore_axis_name="s")
    f = pl.kernel(
        _p4b_body,
        out_type=[jax.ShapeDtypeStruct((_NSUB, _SROW * HD), jnp.float32)] * 2,
        mesh=mesh,
        compiler_params=pltpu.CompilerParams(needs_layout_passes=False),
        scratch_types=[
            pltpu.VMEM((_SROW * HD,), jnp.float32),
            pltpu.VMEM((1, _C), jnp.int32),
            pltpu.VMEM((_C * HEADS,), jnp.float32),
        ],
    )
    return f(dst, pflat)


def _sum32_body(a_ref, b_ref, out_ref):
    out_ref[...] = jnp.sum(a_ref[...], axis=0) + jnp.sum(b_ref[...], axis=0)


def _sum32(t0, t1):
    blk = _SROW * HD // 8
    return pl.pallas_call(
        _sum32_body,
        grid=(8,),
        in_specs=[pl.BlockSpec((_NSUB, blk), lambda i: (0, i))] * 2,
        out_specs=pl.BlockSpec((blk,), lambda i: (i,)),
        out_shape=jax.ShapeDtypeStruct((_SROW * HD,), jnp.float32),
    )(t0, t1)


def _node_body(q_ref, s_ref, av_ref, ac_ref, r8, veb, out_ref):
    inv = 1.0 / (s_ref[...] + 1e-30)
    inv_rep = jnp.dot(inv, r8[...], preferred_element_type=jnp.float32)
    rv = ac_ref[...] * inv_rep
    out_ref[...] = (q_ref[...] + av_ref[...] * inv_rep
                    + jnp.dot(rv, veb[...], preferred_element_type=jnp.float32))


def _node_stage(Q, S, accV, accC, R8, VeBlock):
    full = lambda s: pl.BlockSpec(s, lambda i: (0,) * len(s))
    nspec = pl.BlockSpec((_NB, HD), lambda i: (i, 0))
    return pl.pallas_call(
        _node_body,
        grid=(N // _NB,),
        in_specs=[nspec, pl.BlockSpec((_NB, HEADS), lambda i: (i, 0)),
                  nspec, nspec, full((HEADS, HD)), full((HD, HD))],
        out_specs=nspec,
        out_shape=jax.ShapeDtypeStruct((N, HD), jnp.float32),
    )(Q, S, accV, accC, R8, VeBlock)


def kernel(x, edge_index, edge_attr, WQ, bQ, WK, bK, WE, bE, WV, bV, Aw, VeRow):
    src = edge_index[0]
    dst = edge_index[1]

    h = np.arange(HEADS)
    j = np.arange(H_DIM)
    perm1 = (32 * h[:, None] + j[None, :]).reshape(-1)
    perm2 = (32 * h[:, None] + 16 + j[None, :]).reshape(-1)
    WE1, bE1 = WE[:, perm1], bE[perm1]
    WE2, bE2 = WE[:, perm2], bE[perm2]

    rows = jnp.arange(HD)
    hcol = jnp.repeat(jnp.arange(HEADS), H_DIM)
    AwBlock = jnp.zeros((HD, HEADS), jnp.float32).at[rows, hcol].set(
        Aw[:, :, 0].T.reshape(HD))
    R8 = jnp.zeros((HEADS, HD), jnp.float32).at[hcol, rows].set(1.0)

    h_i = jnp.repeat(jnp.arange(HEADS), H_DIM * H_DIM)
    d_i = jnp.tile(jnp.repeat(jnp.arange(H_DIM), H_DIM), HEADS)
    c_i = jnp.tile(jnp.arange(H_DIM), HEADS * H_DIM)
    VeBlock = jnp.zeros((HD, HD), jnp.float32).at[
        16 * h_i + d_i, 16 * h_i + c_i].set(VeRow[d_i, h_i, c_i])

    K, Q, V = _proj(x, WK, bK, WQ, bQ, WV, bV)

    G = _gather_stage(src, dst, K, Q)

    e_out, p, m2 = _edge_stage(edge_attr, G, WE1, bE1, WE2, bE2, AwBlock, R8)

    zin = jnp.zeros((_NPAD, HD), jnp.float32)
    pflat = p.reshape(-1)
    out0, out1 = _scatter_stage(dst, src, pflat, m2, V, zin)
    outS0, outS1 = _s_stage(dst, pflat)
    accV = out0[:N]
    accC = out1[:N]
    S = _sum32(outS0, outS1).reshape(_NPAD, HEADS)[:N]

    n_out = _node_stage(Q, S, accV, accC, R8, VeBlock)
    return (n_out, e_out)


# speedup vs baseline: 47.4933x; 1.1768x over previous
import functools

import jax
import jax.numpy as jnp
import numpy as np
from jax import lax
from jax.experimental import pallas as pl
from jax.experimental.pallas import tpu as pltpu
from jax.experimental.pallas import tpu_sc as plsc

N = 10000
E_EDGES = 320000
IN_DIM = 128
H_DIM = 16
HEADS = 8
HD = H_DIM * HEADS
CLAMP = 5.0

_NB = 1000
_EB = 2000


def _proj_body(x_ref, wk, bk, wq, bq, wv, bv, k_out, q_out, v_out):
    xb = x_ref[...]
    k_out[...] = jnp.dot(xb, wk[...], preferred_element_type=jnp.float32) + bk[...]
    q_out[...] = jnp.dot(xb, wq[...], preferred_element_type=jnp.float32) + bq[...]
    v_out[...] = jnp.dot(xb, wv[...], preferred_element_type=jnp.float32) + bv[...]


def _proj(x, WK, bK, WQ, bQ, WV, bV):
    full = lambda s: pl.BlockSpec(s, lambda i: (0,) * len(s))
    nspec = pl.BlockSpec((_NB, HD), lambda i: (i, 0))
    return pl.pallas_call(
        _proj_body,
        grid=(N // _NB,),
        in_specs=[pl.BlockSpec((_NB, IN_DIM), lambda i: (i, 0)),
                  full((IN_DIM, HD)), full((1, HD)),
                  full((IN_DIM, HD)), full((1, HD)),
                  full((IN_DIM, HD)), full((1, HD))],
        out_specs=[nspec, nspec, nspec],
        out_shape=[jax.ShapeDtypeStruct((N, HD), jnp.float32)] * 3,
    )(x, WK, bK.reshape(1, HD), WQ, bQ.reshape(1, HD), WV, bV.reshape(1, HD))


def _edge_body(ea_ref, g_ref, we1, be1, we2, be2, awb, r8,
               eout_ref, p_ref, m2_ref):
    ea = ea_ref[...]
    ex1 = jnp.dot(ea, we1[...], preferred_element_type=jnp.float32) + be1[...]
    ex2 = jnp.dot(ea, we2[...], preferred_element_type=jnp.float32) + be2[...]
    s2 = ex1 * ex2
    score2 = jnp.sqrt(jax.nn.relu(s2)) - jnp.sqrt(jax.nn.relu(-s2))
    conn = g_ref[...] + score2
    eout_ref[...] = conn
    score = jnp.dot(conn, awb[...], preferred_element_type=jnp.float32)
    p = jnp.exp(jnp.clip(score, -CLAMP, CLAMP))
    p_ref[...] = p
    prep = jnp.dot(p, r8[...], preferred_element_type=jnp.float32)
    m2_ref[...] = prep * conn


def _edge_stage(edge_attr, G, WE1, bE1, WE2, bE2, AwBlock, R8):
    full = lambda s: pl.BlockSpec(s, lambda i: (0,) * len(s))
    espec = pl.BlockSpec((_EB, HD), lambda i: (i, 0))
    return pl.pallas_call(
        _edge_body,
        grid=(E_EDGES // _EB,),
        in_specs=[espec, espec,
                  full((IN_DIM, HD)), full((1, HD)),
                  full((IN_DIM, HD)), full((1, HD)),
                  full((IN_DIM, HEADS)), full((HEADS, HD))],
        out_specs=[espec, pl.BlockSpec((_EB, HEADS), lambda i: (i, 0)), espec],
        out_shape=[jax.ShapeDtypeStruct((E_EDGES, HD), jnp.float32),
                   jax.ShapeDtypeStruct((E_EDGES, HEADS), jnp.float32),
                   jax.ShapeDtypeStruct((E_EDGES, HD), jnp.float32)],
    )(edge_attr, G, WE1, bE1.reshape(1, HD), WE2, bE2.reshape(1, HD),
      AwBlock, R8)


_C = 128
_NCHUNK = E_EDGES // _C
_NSUB = 16
_NPAD = 10240
_NROW = _NPAD // _NSUB
_SROW = _NPAD // 16
_GDN = lax.GatherDimensionNumbers(
    offset_dims=(), collapsed_slice_dims=(0,), start_index_map=(0,))


def _bcast16(vec, idxvec):
    return lax.gather(vec, idxvec, _GDN, (1,),
                      mode=lax.GatherScatterMode.PROMISE_IN_BOUNDS)


def _p4_body(dstE, srcE, pflat, m2, vN, zin, out0, out1,
             sh, idxb, sidx, pbuf, vbuf, pay, sem):
    cid = lax.axis_index("c")
    sid = lax.axis_index("s")
    iota16 = lax.iota(jnp.int32, 16)
    splat = [(iota16 * 0 + k).reshape(16, 1) for k in range(16)]
    row0 = sid * _NROW
    pltpu.sync_copy(zin.at[pl.ds(row0, _NROW), :], sh.at[pl.ds(row0, _NROW), :])
    plsc.subcore_barrier()

    rem = _NCHUNK - (_NCHUNK // _NSUB) * _NSUB
    nloc = (_NCHUNK // _NSUB) + jnp.where(sid < rem, 1, 0)

    @pl.when(cid == 0)
    def _():
        def body0(i, carry):
            base = (i * _NSUB + sid) * _C
            pltpu.sync_copy(dstE.at[pl.ds(base, _C)], idxb.at[0])
            pltpu.sync_copy(srcE.at[pl.ds(base, _C)], sidx)
            pltpu.sync_copy(pflat.at[pl.ds(base * HEADS, _C * HEADS)], pbuf)
            pltpu.async_copy(vN.at[sidx], vbuf, sem).wait()
            for q2 in range(_C // 2):
                pp = pbuf[pl.ds(q2 * 16, 16)]
                for a in range(2):
                    r = 2 * q2 + a
                    for h in range(HEADS):
                        w = _bcast16(pp, splat[a * HEADS + h])
                        pay[r, pl.ds(h * H_DIM, 16)] = (
                            vbuf[r, pl.ds(h * H_DIM, 16)] * w)
            pltpu.sync_copy(pay, sh.at[idxb.at[0]], add=True)
            return carry
        lax.fori_loop(0, nloc, body0, 0)

    @pl.when(cid == 1)
    def _():
        def body1(i, carry):
            base = (i * _NSUB + sid) * _C
            pltpu.sync_copy(dstE.at[pl.ds(base, _C)], idxb.at[0])
            pltpu.sync_copy(m2.at[pl.ds(base, _C), :], pay)
            pltpu.sync_copy(pay, sh.at[idxb.at[0]], add=True)
            return carry
        lax.fori_loop(0, nloc, body1, 0)

    plsc.subcore_barrier()

    @pl.when(cid == 0)
    def _():
        pltpu.sync_copy(sh.at[pl.ds(row0, _NROW), :],
                        out0.at[pl.ds(row0, _NROW), :])

    @pl.when(cid == 1)
    def _():
        pltpu.sync_copy(sh.at[pl.ds(row0, _NROW), :],
                        out1.at[pl.ds(row0, _NROW), :])


def _scatter_stage(dst, src, pflat, m2, V, zin):
    mesh = plsc.VectorSubcoreMesh(core_axis_name="c", subcore_axis_name="s")
    f = pl.kernel(
        _p4_body,
        out_type=[jax.ShapeDtypeStruct((_NPAD, HD), jnp.float32),
                  jax.ShapeDtypeStruct((_NPAD, HD), jnp.float32)],
        mesh=mesh,
        compiler_params=pltpu.CompilerParams(needs_layout_passes=False),
        scratch_types=[
            pltpu.VMEM_SHARED((_NPAD, HD), jnp.float32),
            pltpu.VMEM((1, _C), jnp.int32),
            pltpu.VMEM((_C,), jnp.int32),
            pltpu.VMEM((_C * HEADS,), jnp.float32),
            pltpu.VMEM((_C, HD), jnp.float32),
            pltpu.VMEM((_C, HD), jnp.float32),
            pltpu.SemaphoreType.DMA,
        ],
    )
    return f(dst, src, pflat, m2, V, zin)


_GC = 64
_GPW = E_EDGES // 32
_GN = _GPW // _GC
_GTAIL = _GPW - _GN * _GC


def _p2_body(srcE, dstE, kN, qN, gE,
             idxs, idxd, bufk, bufq, bufg, ti, tk, tq, isem, gsem, wsem):
    cid = lax.axis_index("c")
    sid = lax.axis_index("s")
    wid = sid * 2 + cid
    w0 = wid * _GPW

    def ebase(ci):
        return w0 + ci * _GC

    def idx_load(ci, b):
        pltpu.async_copy(srcE.at[pl.ds(ebase(ci), _GC)], idxs.at[b], isem)
        pltpu.async_copy(dstE.at[pl.ds(ebase(ci), _GC)], idxd.at[b], isem)

    def idx_wait(b):
        pltpu.make_async_copy(srcE.at[pl.ds(w0, _GC)], idxs.at[b], isem).wait()
        pltpu.make_async_copy(dstE.at[pl.ds(w0, _GC)], idxd.at[b], isem).wait()

    def gat_issue(b):
        pltpu.async_copy(kN.at[idxs.at[b]], bufk.at[b], gsem)
        pltpu.async_copy(qN.at[idxd.at[b]], bufq.at[b], gsem)

    def gat_wait(b):
        pltpu.make_async_copy(kN.at[idxs.at[b]], bufk.at[b], gsem).wait()
        pltpu.make_async_copy(qN.at[idxd.at[b]], bufq.at[b], gsem).wait()

    def wrt_drain(b):
        pltpu.make_async_copy(bufg.at[b], gE.at[pl.ds(w0, _GC), :],
                              wsem).wait()

    idx_load(0, 0)
    idx_wait(0)
    gat_issue(0)
    idx_load(1, 1)

    def body(i2, carry):
        for b in range(2):
            ci = i2 * 2 + b

            @pl.when(ci + 1 < _GN)
            def _():
                idx_wait(b ^ 1)
                gat_issue(b ^ 1)
            gat_wait(b)

            @pl.when(ci + 2 < _GN)
            def _():
                idx_load(ci + 2, b)

            @pl.when(ci >= 2)
            def _():
                wrt_drain(b)
            for r in range(_GC):
                for h in range(HEADS):
                    sl = pl.ds(h * H_DIM, 16)
                    bufg[b, r, sl] = bufk[b, r, sl] + bufq[b, r, sl]
            pltpu.async_copy(bufg.at[b], gE.at[pl.ds(ebase(ci), _GC), :],
                             wsem)
        return carry
    lax.fori_loop(0, _GN // 2, body, 0)
    wrt_drain(0)
    wrt_drain(1)

    tb = w0 + _GN * _GC
    pltpu.sync_copy(srcE.at[pl.ds(tb, _GTAIL)], ti)
    pltpu.async_copy(kN.at[ti], tk, gsem).wait()
    pltpu.sync_copy(dstE.at[pl.ds(tb, _GTAIL)], ti)
    pltpu.async_copy(qN.at[ti], tq, gsem).wait()
    for r in range(_GTAIL):
        for h in range(HEADS):
            sl = pl.ds(h * H_DIM, 16)
            tk[r, sl] = tk[r, sl] + tq[r, sl]
    pltpu.sync_copy(tk, gE.at[pl.ds(tb, _GTAIL), :])


def _gather_stage(src, dst, K, Q):
    mesh = plsc.VectorSubcoreMesh(core_axis_name="c", subcore_axis_name="s")
    f = pl.kernel(
        _p2_body,
        out_type=[jax.ShapeDtypeStruct((E_EDGES, HD), jnp.float32)],
        mesh=mesh,
        compiler_params=pltpu.CompilerParams(needs_layout_passes=False),
        scratch_types=[
            pltpu.VMEM((2, _GC), jnp.int32),
            pltpu.VMEM((2, _GC), jnp.int32),
            pltpu.VMEM((2, _GC, HD), jnp.float32),
            pltpu.VMEM((2, _GC, HD), jnp.float32),
            pltpu.VMEM((2, _GC, HD), jnp.float32),
            pltpu.VMEM((_GTAIL,), jnp.int32),
            pltpu.VMEM((_GTAIL, HD), jnp.float32),
            pltpu.VMEM((_GTAIL, HD), jnp.float32),
            pltpu.SemaphoreType.DMA,
            pltpu.SemaphoreType.DMA,
            pltpu.SemaphoreType.DMA,
        ],
    )
    return f(src, dst, K, Q)[0]


def _p4b_body(dstE, pflat, outS0, outS1, sloc, idxb, pbuf):
    cid = lax.axis_index("c")
    sid = lax.axis_index("s")
    wid = sid * 2 + cid
    iota16 = lax.iota(jnp.int32, 16)
    splat = [(iota16 * 0 + k).reshape(16, 1) for k in range(16)]
    msk8 = iota16 < 8
    ioff = lax.bitwise_and(iota16, 7)
    zf = iota16.astype(jnp.float32) * 0.0

    def zbody(i, carry):
        for j in range(8):
            sloc[pl.ds(i * 128 + j * 16, 16)] = zf
        return carry
    lax.fori_loop(0, _SROW * HD // 128, zbody, 0)

    nw = _NCHUNK // 32
    rem = _NCHUNK - nw * 32
    nloc = nw + jnp.where(wid < rem, 1, 0)

    def body(i, carry):
        base = (i * 32 + wid) * _C
        pltpu.sync_copy(dstE.at[pl.ds(base, _C)], idxb.at[0])
        pltpu.sync_copy(pflat.at[pl.ds(base * HEADS, _C * HEADS)], pbuf)
        for q2 in range(_C // 2):
            pp = pbuf[pl.ds(q2 * 16, 16)]
            if q2 % 8 == 0:
                dwin = idxb[0, pl.ds(q2 * 2, 16)]
            d0 = _bcast16(dwin, splat[(2 * q2) % 16])
            d1 = _bcast16(dwin, splat[(2 * q2 + 1) % 16])
            plsc.addupdate_scatter(sloc, [d0 * HEADS + ioff], pp, mask=msk8)
            plsc.addupdate_scatter(sloc, [d1 * HEADS + ioff], pp, mask=~msk8)
        return carry
    lax.fori_loop(0, nloc, body, 0)

    @pl.when(cid == 0)
    def _():
        pltpu.sync_copy(sloc, outS0.at[sid])

    @pl.when(cid == 1)
    def _():
        pltpu.sync_copy(sloc, outS1.at[sid])


def _s_stage(dst, pflat):
    mesh = plsc.VectorSubcoreMesh(core_axis_name="c", subcore_axis_name="s")
    f = pl.kernel(
        _p4b_body,
        out_type=[jax.ShapeDtypeStruct((_NSUB, _SROW * HD), jnp.float32)] * 2,
        mesh=mesh,
        compiler_params=pltpu.CompilerParams(needs_layout_passes=False),
        scratch_types=[
            pltpu.VMEM((_SROW * HD,), jnp.float32),
            pltpu.VMEM((1, _C), jnp.int32),
            pltpu.VMEM((_C * HEADS,), jnp.float32),
        ],
    )
    return f(dst, pflat)


def _sum32_body(a_ref, b_ref, out_ref):
    out_ref[...] = jnp.sum(a_ref[...], axis=0) + jnp.sum(b_ref[...], axis=0)


def _sum32(t0, t1):
    blk = _SROW * HD // 8
    return pl.pallas_call(
        _sum32_body,
        grid=(8,),
        in_specs=[pl.BlockSpec((_NSUB, blk), lambda i: (0, i))] * 2,
        out_specs=pl.BlockSpec((blk,), lambda i: (i,)),
        out_shape=jax.ShapeDtypeStruct((_SROW * HD,), jnp.float32),
    )(t0, t1)


def _node_body(q_ref, s_ref, av_ref, ac_ref, r8, veb, out_ref):
    inv = 1.0 / (s_ref[...] + 1e-30)
    inv_rep = jnp.dot(inv, r8[...], preferred_element_type=jnp.float32)
    rv = ac_ref[...] * inv_rep
    out_ref[...] = (q_ref[...] + av_ref[...] * inv_rep
                    + jnp.dot(rv, veb[...], preferred_element_type=jnp.float32))


def _node_stage(Q, S, accV, accC, R8, VeBlock):
    full = lambda s: pl.BlockSpec(s, lambda i: (0,) * len(s))
    nspec = pl.BlockSpec((_NB, HD), lambda i: (i, 0))
    return pl.pallas_call(
        _node_body,
        grid=(N // _NB,),
        in_specs=[nspec, pl.BlockSpec((_NB, HEADS), lambda i: (i, 0)),
                  nspec, nspec, full((HEADS, HD)), full((HD, HD))],
        out_specs=nspec,
        out_shape=jax.ShapeDtypeStruct((N, HD), jnp.float32),
    )(Q, S, accV, accC, R8, VeBlock)


def kernel(x, edge_index, edge_attr, WQ, bQ, WK, bK, WE, bE, WV, bV, Aw, VeRow):
    src = edge_index[0]
    dst = edge_index[1]

    h = np.arange(HEADS)
    j = np.arange(H_DIM)
    perm1 = (32 * h[:, None] + j[None, :]).reshape(-1)
    perm2 = (32 * h[:, None] + 16 + j[None, :]).reshape(-1)
    WE1, bE1 = WE[:, perm1], bE[perm1]
    WE2, bE2 = WE[:, perm2], bE[perm2]

    rows = jnp.arange(HD)
    hcol = jnp.repeat(jnp.arange(HEADS), H_DIM)
    AwBlock = jnp.zeros((HD, HEADS), jnp.float32).at[rows, hcol].set(
        Aw[:, :, 0].T.reshape(HD))
    R8 = jnp.zeros((HEADS, HD), jnp.float32).at[hcol, rows].set(1.0)

    h_i = jnp.repeat(jnp.arange(HEADS), H_DIM * H_DIM)
    d_i = jnp.tile(jnp.repeat(jnp.arange(H_DIM), H_DIM), HEADS)
    c_i = jnp.tile(jnp.arange(H_DIM), HEADS * H_DIM)
    VeBlock = jnp.zeros((HD, HD), jnp.float32).at[
        16 * h_i + d_i, 16 * h_i + c_i].set(VeRow[d_i, h_i, c_i])

    K, Q, V = _proj(x, WK, bK, WQ, bQ, WV, bV)

    G = _gather_stage(src, dst, K, Q)

    e_out, p, m2 = _edge_stage(edge_attr, G, WE1, bE1, WE2, bE2, AwBlock, R8)

    zin = jnp.zeros((_NPAD, HD), jnp.float32)
    pflat = p.reshape(-1)
    out0, out1 = _scatter_stage(dst, src, pflat, m2, V, zin)
    outS0, outS1 = _s_stage(dst, pflat)
    accV = out0[:N]
    accC = out1[:N]
    S = _sum32(outS0, outS1).reshape(_NPAD, HEADS)[:N]

    n_out = _node_stage(Q, S, accV, accC, R8, VeBlock)
    return (n_out, e_out)


# speedup vs baseline: 52.7586x; 1.1109x over previous
import functools

import jax
import jax.numpy as jnp
import numpy as np
from jax import lax
from jax.experimental import pallas as pl
from jax.experimental.pallas import tpu as pltpu
from jax.experimental.pallas import tpu_sc as plsc

N = 10000
E_EDGES = 320000
IN_DIM = 128
H_DIM = 16
HEADS = 8
HD = H_DIM * HEADS
CLAMP = 5.0

_NB = 1000
_EB = 2000


def _proj_body(x_ref, wk, bk, wq, bq, wv, bv, k_out, q_out, v_out):
    xb = x_ref[...]
    k_out[...] = jnp.dot(xb, wk[...], preferred_element_type=jnp.float32) + bk[...]
    q_out[...] = jnp.dot(xb, wq[...], preferred_element_type=jnp.float32) + bq[...]
    v_out[...] = jnp.dot(xb, wv[...], preferred_element_type=jnp.float32) + bv[...]


def _proj(x, WK, bK, WQ, bQ, WV, bV):
    full = lambda s: pl.BlockSpec(s, lambda i: (0,) * len(s))
    nspec = pl.BlockSpec((_NB, HD), lambda i: (i, 0))
    return pl.pallas_call(
        _proj_body,
        grid=(N // _NB,),
        in_specs=[pl.BlockSpec((_NB, IN_DIM), lambda i: (i, 0)),
                  full((IN_DIM, HD)), full((1, HD)),
                  full((IN_DIM, HD)), full((1, HD)),
                  full((IN_DIM, HD)), full((1, HD))],
        out_specs=[nspec, nspec, nspec],
        out_shape=[jax.ShapeDtypeStruct((N, HD), jnp.float32)] * 3,
    )(x, WK, bK.reshape(1, HD), WQ, bQ.reshape(1, HD), WV, bV.reshape(1, HD))


def _edge_body(ea_ref, g_ref, we1, be1, we2, be2, awb, r8,
               eout_ref, p_ref, m2_ref):
    ea = ea_ref[...]
    ex1 = jnp.dot(ea, we1[...], preferred_element_type=jnp.float32) + be1[...]
    ex2 = jnp.dot(ea, we2[...], preferred_element_type=jnp.float32) + be2[...]
    s2 = ex1 * ex2
    score2 = jnp.sqrt(jax.nn.relu(s2)) - jnp.sqrt(jax.nn.relu(-s2))
    conn = g_ref[...] + score2
    eout_ref[...] = conn
    score = jnp.dot(conn, awb[...], preferred_element_type=jnp.float32)
    p = jnp.exp(jnp.clip(score, -CLAMP, CLAMP))
    p_ref[...] = p
    prep = jnp.dot(p, r8[...], preferred_element_type=jnp.float32)
    m2_ref[...] = prep * conn


def _edge_stage(edge_attr, G, WE1, bE1, WE2, bE2, AwBlock, R8):
    full = lambda s: pl.BlockSpec(s, lambda i: (0,) * len(s))
    espec = pl.BlockSpec((_EB, HD), lambda i: (i, 0))
    return pl.pallas_call(
        _edge_body,
        grid=(E_EDGES // _EB,),
        in_specs=[espec, espec,
                  full((IN_DIM, HD)), full((1, HD)),
                  full((IN_DIM, HD)), full((1, HD)),
                  full((IN_DIM, HEADS)), full((HEADS, HD))],
        out_specs=[espec, pl.BlockSpec((_EB, HEADS), lambda i: (i, 0)), espec],
        out_shape=[jax.ShapeDtypeStruct((E_EDGES, HD), jnp.float32),
                   jax.ShapeDtypeStruct((E_EDGES, HEADS), jnp.float32),
                   jax.ShapeDtypeStruct((E_EDGES, HD), jnp.float32)],
    )(edge_attr, G, WE1, bE1.reshape(1, HD), WE2, bE2.reshape(1, HD),
      AwBlock, R8)


_C = 128
_NCHUNK = E_EDGES // _C
_NSUB = 16
_NPAD = 10240
_NROW = _NPAD // _NSUB
_SROW = _NPAD // 16
_GDN = lax.GatherDimensionNumbers(
    offset_dims=(), collapsed_slice_dims=(0,), start_index_map=(0,))


def _bcast16(vec, idxvec):
    return lax.gather(vec, idxvec, _GDN, (1,),
                      mode=lax.GatherScatterMode.PROMISE_IN_BOUNDS)


_SCW = E_EDGES // _NSUB
_SC4 = 32
_SNT = _SCW // _SC4
_SNL = (_SNT - 1) // 4


def _p4_body(dstE, srcE, pflat, m2, vN, zin, out0, out1,
             sh, didx, sidx, pbuf, vbuf, pay, mbuf, isem, gsem, ssem):
    cid = lax.axis_index("c")
    sid = lax.axis_index("s")
    iota16 = lax.iota(jnp.int32, 16)
    splat = [(iota16 * 0 + k).reshape(16, 1) for k in range(16)]
    row0 = sid * _NROW
    pltpu.sync_copy(zin.at[pl.ds(row0, _NROW), :], sh.at[pl.ds(row0, _NROW), :])
    plsc.subcore_barrier()

    e0 = sid * _SCW

    def ebase(ci):
        return e0 + ci * _SC4

    def idx_load0(ci):
        b, m = ci % 2, ci % 4
        pltpu.async_copy(srcE.at[pl.ds(ebase(ci), _SC4)],
                         sidx.at[pl.ds(b * _SC4, _SC4)], isem)
        pltpu.async_copy(pflat.at[pl.ds(ebase(ci) * HEADS, _SC4 * HEADS)],
                         pbuf.at[pl.ds(b * _SC4 * HEADS, _SC4 * HEADS)], isem)
        pltpu.async_copy(dstE.at[pl.ds(ebase(ci), _SC4)], didx.at[m], isem)

    def idx_wait0(ci):
        b, m = ci % 2, ci % 4
        pltpu.make_async_copy(srcE.at[pl.ds(e0, _SC4)],
                              sidx.at[pl.ds(b * _SC4, _SC4)], isem).wait()
        pltpu.make_async_copy(pflat.at[pl.ds(e0, _SC4 * HEADS)],
                              pbuf.at[pl.ds(b * _SC4 * HEADS, _SC4 * HEADS)],
                              isem).wait()
        pltpu.make_async_copy(dstE.at[pl.ds(e0, _SC4)], didx.at[m],
                              isem).wait()

    def scat_drain0(ci):
        b, m = ci % 2, ci % 4
        pltpu.make_async_copy(pay.at[b], sh.at[didx.at[m]], ssem).wait()

    def substep0(ci, b, m):
        @pl.when(ci + 1 < _SNT)
        def _():
            idx_wait0(ci + 1)
            pltpu.async_copy(vN.at[sidx.at[pl.ds((b ^ 1) * _SC4, _SC4)]],
                             vbuf.at[b ^ 1], gsem)
        pltpu.make_async_copy(vN.at[sidx.at[pl.ds(b * _SC4, _SC4)]],
                              vbuf.at[b], gsem).wait()

        @pl.when(ci >= 2)
        def _():
            scat_drain0(ci - 2)
        for q2 in range(_SC4 // 2):
            pp = pbuf[pl.ds(b * _SC4 * HEADS + q2 * 16, 16)]
            for a in range(2):
                r = 2 * q2 + a
                for h in range(HEADS):
                    w = _bcast16(pp, splat[a * HEADS + h])
                    pay[b, r, pl.ds(h * H_DIM, 16)] = (
                        vbuf[b, r, pl.ds(h * H_DIM, 16)] * w)
        pltpu.async_copy(pay.at[b], sh.at[didx.at[m]], ssem, add=True)

        @pl.when(ci + 2 < _SNT)
        def _():
            idx_load0(ci + 2)

    @pl.when(cid == 0)
    def _():
        idx_load0(0)
        idx_load0(1)
        idx_wait0(0)
        pltpu.async_copy(vN.at[sidx.at[pl.ds(0, _SC4)]], vbuf.at[0], gsem)

        def body0(i4, carry):
            for j in range(4):
                substep0(i4 * 4 + j, j % 2, j)
            return carry
        lax.fori_loop(0, _SNL, body0, 0)
        for j in range(4):
            if _SNL * 4 + j < _SNT:
                substep0(_SNL * 4 + j, j % 2, j)
        scat_drain0(_SNT - 2)
        scat_drain0(_SNT - 1)

    def mload1(ci):
        m = ci % 4
        pltpu.async_copy(dstE.at[pl.ds(ebase(ci), _SC4)], didx.at[m], isem)
        pltpu.async_copy(m2.at[pl.ds(ebase(ci), _SC4), :], mbuf.at[m], isem)

    def mwait1(ci):
        m = ci % 4
        pltpu.make_async_copy(dstE.at[pl.ds(e0, _SC4)], didx.at[m],
                              isem).wait()
        pltpu.make_async_copy(m2.at[pl.ds(e0, _SC4), :], mbuf.at[m],
                              isem).wait()

    def scat_drain1(ci):
        m = ci % 4
        pltpu.make_async_copy(mbuf.at[m], sh.at[didx.at[m]], ssem).wait()

    def substep1(ci, m):
        mwait1(ci)

        @pl.when(ci >= 2)
        def _():
            scat_drain1(ci - 2)
        pltpu.async_copy(mbuf.at[m], sh.at[didx.at[m]], ssem, add=True)

        @pl.when(ci + 2 < _SNT)
        def _():
            mload1(ci + 2)

    @pl.when(cid == 1)
    def _():
        mload1(0)
        mload1(1)

        def body1(i4, carry):
            for j in range(4):
                substep1(i4 * 4 + j, j)
            return carry
        lax.fori_loop(0, _SNL, body1, 0)
        for j in range(4):
            if _SNL * 4 + j < _SNT:
                substep1(_SNL * 4 + j, j)
        scat_drain1(_SNT - 2)
        scat_drain1(_SNT - 1)

    plsc.subcore_barrier()

    @pl.when(cid == 0)
    def _():
        pltpu.sync_copy(sh.at[pl.ds(row0, _NROW), :],
                        out0.at[pl.ds(row0, _NROW), :])

    @pl.when(cid == 1)
    def _():
        pltpu.sync_copy(sh.at[pl.ds(row0, _NROW), :],
                        out1.at[pl.ds(row0, _NROW), :])


def _scatter_stage(dst, src, pflat, m2, V, zin):
    mesh = plsc.VectorSubcoreMesh(core_axis_name="c", subcore_axis_name="s")
    f = pl.kernel(
        _p4_body,
        out_type=[jax.ShapeDtypeStruct((_NPAD, HD), jnp.float32),
                  jax.ShapeDtypeStruct((_NPAD, HD), jnp.float32)],
        mesh=mesh,
        compiler_params=pltpu.CompilerParams(needs_layout_passes=False),
        scratch_types=[
            pltpu.VMEM_SHARED((_NPAD, HD), jnp.float32),
            pltpu.VMEM((4, _SC4), jnp.int32),
            pltpu.VMEM((2 * _SC4,), jnp.int32),
            pltpu.VMEM((2 * _SC4 * HEADS,), jnp.float32),
            pltpu.VMEM((2, _SC4, HD), jnp.float32),
            pltpu.VMEM((2, _SC4, HD), jnp.float32),
            pltpu.VMEM((4, _SC4, HD), jnp.float32),
            pltpu.SemaphoreType.DMA,
            pltpu.SemaphoreType.DMA,
            pltpu.SemaphoreType.DMA,
        ],
    )
    return f(dst, src, pflat, m2, V, zin)


_GC = 64
_GPW = E_EDGES // 32
_GN = _GPW // _GC
_GTAIL = _GPW - _GN * _GC


def _p2_body(srcE, dstE, kN, qN, gE,
             idxs, idxd, bufk, bufq, bufg, ti, tk, tq, isem, gsem, wsem):
    cid = lax.axis_index("c")
    sid = lax.axis_index("s")
    wid = sid * 2 + cid
    w0 = wid * _GPW

    def ebase(ci):
        return w0 + ci * _GC

    def idx_load(ci, b):
        pltpu.async_copy(srcE.at[pl.ds(ebase(ci), _GC)], idxs.at[b], isem)
        pltpu.async_copy(dstE.at[pl.ds(ebase(ci), _GC)], idxd.at[b], isem)

    def idx_wait(b):
        pltpu.make_async_copy(srcE.at[pl.ds(w0, _GC)], idxs.at[b], isem).wait()
        pltpu.make_async_copy(dstE.at[pl.ds(w0, _GC)], idxd.at[b], isem).wait()

    def gat_issue(b):
        pltpu.async_copy(kN.at[idxs.at[b]], bufk.at[b], gsem)
        pltpu.async_copy(qN.at[idxd.at[b]], bufq.at[b], gsem)

    def gat_wait(b):
        pltpu.make_async_copy(kN.at[idxs.at[b]], bufk.at[b], gsem).wait()
        pltpu.make_async_copy(qN.at[idxd.at[b]], bufq.at[b], gsem).wait()

    def wrt_drain(b):
        pltpu.make_async_copy(bufg.at[b], gE.at[pl.ds(w0, _GC), :],
                              wsem).wait()

    idx_load(0, 0)
    idx_wait(0)
    gat_issue(0)
    idx_load(1, 1)

    def body(i2, carry):
        for b in range(2):
            ci = i2 * 2 + b

            @pl.when(ci + 1 < _GN)
            def _():
                idx_wait(b ^ 1)
                gat_issue(b ^ 1)
            gat_wait(b)

            @pl.when(ci + 2 < _GN)
            def _():
                idx_load(ci + 2, b)

            @pl.when(ci >= 2)
            def _():
                wrt_drain(b)
            for r in range(_GC):
                for h in range(HEADS):
                    sl = pl.ds(h * H_DIM, 16)
                    bufg[b, r, sl] = bufk[b, r, sl] + bufq[b, r, sl]
            pltpu.async_copy(bufg.at[b], gE.at[pl.ds(ebase(ci), _GC), :],
                             wsem)
        return carry
    lax.fori_loop(0, _GN // 2, body, 0)
    wrt_drain(0)
    wrt_drain(1)

    tb = w0 + _GN * _GC
    pltpu.sync_copy(srcE.at[pl.ds(tb, _GTAIL)], ti)
    pltpu.async_copy(kN.at[ti], tk, gsem).wait()
    pltpu.sync_copy(dstE.at[pl.ds(tb, _GTAIL)], ti)
    pltpu.async_copy(qN.at[ti], tq, gsem).wait()
    for r in range(_GTAIL):
        for h in range(HEADS):
            sl = pl.ds(h * H_DIM, 16)
            tk[r, sl] = tk[r, sl] + tq[r, sl]
    pltpu.sync_copy(tk, gE.at[pl.ds(tb, _GTAIL), :])


def _gather_stage(src, dst, K, Q):
    mesh = plsc.VectorSubcoreMesh(core_axis_name="c", subcore_axis_name="s")
    f = pl.kernel(
        _p2_body,
        out_type=[jax.ShapeDtypeStruct((E_EDGES, HD), jnp.float32)],
        mesh=mesh,
        compiler_params=pltpu.CompilerParams(needs_layout_passes=False),
        scratch_types=[
            pltpu.VMEM((2, _GC), jnp.int32),
            pltpu.VMEM((2, _GC), jnp.int32),
            pltpu.VMEM((2, _GC, HD), jnp.float32),
            pltpu.VMEM((2, _GC, HD), jnp.float32),
            pltpu.VMEM((2, _GC, HD), jnp.float32),
            pltpu.VMEM((_GTAIL,), jnp.int32),
            pltpu.VMEM((_GTAIL, HD), jnp.float32),
            pltpu.VMEM((_GTAIL, HD), jnp.float32),
            pltpu.SemaphoreType.DMA,
            pltpu.SemaphoreType.DMA,
            pltpu.SemaphoreType.DMA,
        ],
    )
    return f(src, dst, K, Q)[0]


def _p4b_body(dstE, pflat, outS0, outS1, sloc, idxb, pbuf):
    cid = lax.axis_index("c")
    sid = lax.axis_index("s")
    wid = sid * 2 + cid
    iota16 = lax.iota(jnp.int32, 16)
    splat = [(iota16 * 0 + k).reshape(16, 1) for k in range(16)]
    msk8 = iota16 < 8
    ioff = lax.bitwise_and(iota16, 7)
    zf = iota16.astype(jnp.float32) * 0.0

    def zbody(i, carry):
        for j in range(8):
            sloc[pl.ds(i * 128 + j * 16, 16)] = zf
        return carry
    lax.fori_loop(0, _SROW * HD // 128, zbody, 0)

    nw = _NCHUNK // 32
    rem = _NCHUNK - nw * 32
    nloc = nw + jnp.where(wid < rem, 1, 0)

    def body(i, carry):
        base = (i * 32 + wid) * _C
        pltpu.sync_copy(dstE.at[pl.ds(base, _C)], idxb.at[0])
        pltpu.sync_copy(pflat.at[pl.ds(base * HEADS, _C * HEADS)], pbuf)
        for q2 in range(_C // 2):
            pp = pbuf[pl.ds(q2 * 16, 16)]
            if q2 % 8 == 0:
                dwin = idxb[0, pl.ds(q2 * 2, 16)]
            d0 = _bcast16(dwin, splat[(2 * q2) % 16])
            d1 = _bcast16(dwin, splat[(2 * q2 + 1) % 16])
            plsc.addupdate_scatter(sloc, [d0 * HEADS + ioff], pp, mask=msk8)
            plsc.addupdate_scatter(sloc, [d1 * HEADS + ioff], pp, mask=~msk8)
        return carry
    lax.fori_loop(0, nloc, body, 0)

    @pl.when(cid == 0)
    def _():
        pltpu.sync_copy(sloc, outS0.at[sid])

    @pl.when(cid == 1)
    def _():
        pltpu.sync_copy(sloc, outS1.at[sid])


def _s_stage(dst, pflat):
    mesh = plsc.VectorSubcoreMesh(core_axis_name="c", subcore_axis_name="s")
    f = pl.kernel(
        _p4b_body,
        out_type=[jax.ShapeDtypeStruct((_NSUB, _SROW * HD), jnp.float32)] * 2,
        mesh=mesh,
        compiler_params=pltpu.CompilerParams(needs_layout_passes=False),
        scratch_types=[
            pltpu.VMEM((_SROW * HD,), jnp.float32),
            pltpu.VMEM((1, _C), jnp.int32),
            pltpu.VMEM((_C * HEADS,), jnp.float32),
        ],
    )
    return f(dst, pflat)


def _sum32_body(a_ref, b_ref, out_ref):
    out_ref[...] = jnp.sum(a_ref[...], axis=0) + jnp.sum(b_ref[...], axis=0)


def _sum32(t0, t1):
    blk = _SROW * HD // 8
    return pl.pallas_call(
        _sum32_body,
        grid=(8,),
        in_specs=[pl.BlockSpec((_NSUB, blk), lambda i: (0, i))] * 2,
        out_specs=pl.BlockSpec((blk,), lambda i: (i,)),
        out_shape=jax.ShapeDtypeStruct((_SROW * HD,), jnp.float32),
    )(t0, t1)


def _node_body(q_ref, s_ref, av_ref, ac_ref, r8, veb, out_ref):
    inv = 1.0 / (s_ref[...] + 1e-30)
    inv_rep = jnp.dot(inv, r8[...], preferred_element_type=jnp.float32)
    rv = ac_ref[...] * inv_rep
    out_ref[...] = (q_ref[...] + av_ref[...] * inv_rep
                    + jnp.dot(rv, veb[...], preferred_element_type=jnp.float32))


def _node_stage(Q, S, accV, accC, R8, VeBlock):
    full = lambda s: pl.BlockSpec(s, lambda i: (0,) * len(s))
    nspec = pl.BlockSpec((_NB, HD), lambda i: (i, 0))
    return pl.pallas_call(
        _node_body,
        grid=(N // _NB,),
        in_specs=[nspec, pl.BlockSpec((_NB, HEADS), lambda i: (i, 0)),
                  nspec, nspec, full((HEADS, HD)), full((HD, HD))],
        out_specs=nspec,
        out_shape=jax.ShapeDtypeStruct((N, HD), jnp.float32),
    )(Q, S, accV, accC, R8, VeBlock)


def kernel(x, edge_index, edge_attr, WQ, bQ, WK, bK, WE, bE, WV, bV, Aw, VeRow):
    src = edge_index[0]
    dst = edge_index[1]

    h = np.arange(HEADS)
    j = np.arange(H_DIM)
    perm1 = (32 * h[:, None] + j[None, :]).reshape(-1)
    perm2 = (32 * h[:, None] + 16 + j[None, :]).reshape(-1)
    WE1, bE1 = WE[:, perm1], bE[perm1]
    WE2, bE2 = WE[:, perm2], bE[perm2]

    rows = jnp.arange(HD)
    hcol = jnp.repeat(jnp.arange(HEADS), H_DIM)
    AwBlock = jnp.zeros((HD, HEADS), jnp.float32).at[rows, hcol].set(
        Aw[:, :, 0].T.reshape(HD))
    R8 = jnp.zeros((HEADS, HD), jnp.float32).at[hcol, rows].set(1.0)

    h_i = jnp.repeat(jnp.arange(HEADS), H_DIM * H_DIM)
    d_i = jnp.tile(jnp.repeat(jnp.arange(H_DIM), H_DIM), HEADS)
    c_i = jnp.tile(jnp.arange(H_DIM), HEADS * H_DIM)
    VeBlock = jnp.zeros((HD, HD), jnp.float32).at[
        16 * h_i + d_i, 16 * h_i + c_i].set(VeRow[d_i, h_i, c_i])

    K, Q, V = _proj(x, WK, bK, WQ, bQ, WV, bV)

    G = _gather_stage(src, dst, K, Q)

    e_out, p, m2 = _edge_stage(edge_attr, G, WE1, bE1, WE2, bE2, AwBlock, R8)

    zin = jnp.zeros((_NPAD, HD), jnp.float32)
    pflat = p.reshape(-1)
    out0, out1 = _scatter_stage(dst, src, pflat, m2, V, zin)
    outS0, outS1 = _s_stage(dst, pflat)
    accV = out0[:N]
    accC = out1[:N]
    S = _sum32(outS0, outS1).reshape(_NPAD, HEADS)[:N]

    n_out = _node_stage(Q, S, accV, accC, R8, VeBlock)
    return (n_out, e_out)


# speedup vs baseline: 61.0302x; 1.1568x over previous
import functools

import jax
import jax.numpy as jnp
import numpy as np
from jax import lax
from jax.experimental import pallas as pl
from jax.experimental.pallas import tpu as pltpu
from jax.experimental.pallas import tpu_sc as plsc

N = 10000
E_EDGES = 320000
IN_DIM = 128
H_DIM = 16
HEADS = 8
HD = H_DIM * HEADS
CLAMP = 5.0

_NB = 1000
_EB = 2000


def _proj_body(x_ref, wk, bk, wq, bq, wv, bv, k_out, q_out, v_out):
    xb = x_ref[...]
    k_out[...] = jnp.dot(xb, wk[...], preferred_element_type=jnp.float32) + bk[...]
    q_out[...] = jnp.dot(xb, wq[...], preferred_element_type=jnp.float32) + bq[...]
    v_out[...] = jnp.dot(xb, wv[...], preferred_element_type=jnp.float32) + bv[...]


def _proj(x, WK, bK, WQ, bQ, WV, bV):
    full = lambda s: pl.BlockSpec(s, lambda i: (0,) * len(s))
    nspec = pl.BlockSpec((_NB, HD), lambda i: (i, 0))
    return pl.pallas_call(
        _proj_body,
        grid=(N // _NB,),
        in_specs=[pl.BlockSpec((_NB, IN_DIM), lambda i: (i, 0)),
                  full((IN_DIM, HD)), full((1, HD)),
                  full((IN_DIM, HD)), full((1, HD)),
                  full((IN_DIM, HD)), full((1, HD))],
        out_specs=[nspec, nspec, nspec],
        out_shape=[jax.ShapeDtypeStruct((N, HD), jnp.float32)] * 3,
    )(x, WK, bK.reshape(1, HD), WQ, bQ.reshape(1, HD), WV, bV.reshape(1, HD))


def _edge_body(ea_ref, g_ref, we1, be1, we2, be2, awb, r8,
               eout_ref, p_ref, m2_ref):
    ea = ea_ref[...]
    ex1 = jnp.dot(ea, we1[...], preferred_element_type=jnp.float32) + be1[...]
    ex2 = jnp.dot(ea, we2[...], preferred_element_type=jnp.float32) + be2[...]
    s2 = ex1 * ex2
    score2 = jnp.sqrt(jax.nn.relu(s2)) - jnp.sqrt(jax.nn.relu(-s2))
    conn = g_ref[...] + score2
    eout_ref[...] = conn
    score = jnp.dot(conn, awb[...], preferred_element_type=jnp.float32)
    p = jnp.exp(jnp.clip(score, -CLAMP, CLAMP))
    p_ref[...] = p
    prep = jnp.dot(p, r8[...], preferred_element_type=jnp.float32)
    m2_ref[...] = prep * conn


def _edge_stage(edge_attr, G, WE1, bE1, WE2, bE2, AwBlock, R8):
    full = lambda s: pl.BlockSpec(s, lambda i: (0,) * len(s))
    espec = pl.BlockSpec((_EB, HD), lambda i: (i, 0))
    return pl.pallas_call(
        _edge_body,
        grid=(E_EDGES // _EB,),
        in_specs=[espec, espec,
                  full((IN_DIM, HD)), full((1, HD)),
                  full((IN_DIM, HD)), full((1, HD)),
                  full((IN_DIM, HEADS)), full((HEADS, HD))],
        out_specs=[espec, pl.BlockSpec((_EB, HEADS), lambda i: (i, 0)), espec],
        out_shape=[jax.ShapeDtypeStruct((E_EDGES, HD), jnp.float32),
                   jax.ShapeDtypeStruct((E_EDGES, HEADS), jnp.float32),
                   jax.ShapeDtypeStruct((E_EDGES, HD), jnp.float32)],
    )(edge_attr, G, WE1, bE1.reshape(1, HD), WE2, bE2.reshape(1, HD),
      AwBlock, R8)


_C = 128
_NCHUNK = E_EDGES // _C
_NSUB = 16
_NPAD = 10240
_NROW = _NPAD // _NSUB
_SROW = _NPAD // 16
_GDN = lax.GatherDimensionNumbers(
    offset_dims=(), collapsed_slice_dims=(0,), start_index_map=(0,))


def _bcast16(vec, idxvec):
    return lax.gather(vec, idxvec, _GDN, (1,),
                      mode=lax.GatherScatterMode.PROMISE_IN_BOUNDS)


_SCW = E_EDGES // _NSUB
_SC4 = 80
_SNT = _SCW // _SC4


def _p4_body(dstE, srcE, pflat, m2, vN, zin, out0, out1,
             sh, didx0, didx1, sidx, pbuf, vbuf, pay, isem, gsem, ssem):
    cid = lax.axis_index("c")
    sid = lax.axis_index("s")
    iota16 = lax.iota(jnp.int32, 16)
    splat = [(iota16 * 0 + k).reshape(16, 1) for k in range(16)]
    row0 = sid * _NROW
    pltpu.sync_copy(zin.at[pl.ds(row0, _NROW), :], sh.at[pl.ds(row0, _NROW), :])
    plsc.subcore_barrier()

    e0 = sid * _SCW

    def ebase(ci):
        return e0 + ci * _SC4

    def sp_load0(ci):
        b = ci % 2
        pltpu.async_copy(srcE.at[pl.ds(ebase(ci), _SC4)],
                         sidx.at[pl.ds(b * _SC4, _SC4)], isem)
        pltpu.async_copy(pflat.at[pl.ds(ebase(ci) * HEADS, _SC4 * HEADS)],
                         pbuf.at[pl.ds(b * _SC4 * HEADS, _SC4 * HEADS)], isem)

    def sp_wait0(ci):
        b = ci % 2
        pltpu.make_async_copy(srcE.at[pl.ds(e0, _SC4)],
                              sidx.at[pl.ds(b * _SC4, _SC4)], isem).wait()
        pltpu.make_async_copy(pflat.at[pl.ds(e0, _SC4 * HEADS)],
                              pbuf.at[pl.ds(b * _SC4 * HEADS, _SC4 * HEADS)],
                              isem).wait()

    def vgat_issue0(ci):
        b = ci % 2
        pltpu.async_copy(vN.at[sidx.at[pl.ds(b * _SC4, _SC4)]], vbuf.at[b],
                         gsem)

    def vgat_wait0(ci):
        b = ci % 2
        pltpu.make_async_copy(vN.at[sidx.at[pl.ds(b * _SC4, _SC4)]],
                              vbuf.at[b], gsem).wait()

    def didx_load0(ci):
        pltpu.async_copy(dstE.at[pl.ds(ebase(ci), _SC4)], didx0.at[ci % 2],
                         isem)

    def didx_wait0(ci):
        pltpu.make_async_copy(dstE.at[pl.ds(e0, _SC4)], didx0.at[ci % 2],
                              isem).wait()

    def scat_drain0(ci):
        b = ci % 2
        pltpu.make_async_copy(pay.at[b], sh.at[didx0.at[b]], ssem).wait()

    def substep0(ci, b):
        @pl.when(ci + 1 < _SNT)
        def _():
            sp_wait0(ci + 1)
            vgat_issue0(ci + 1)
        vgat_wait0(ci)
        for q2 in range(_SC4 // 2):
            pp = pbuf[pl.ds(b * _SC4 * HEADS + q2 * 16, 16)]
            for a in range(2):
                r = 2 * q2 + a
                for h in range(HEADS):
                    w = _bcast16(pp, splat[a * HEADS + h])
                    pay[b, r, pl.ds(h * H_DIM, 16)] = (
                        vbuf[b, r, pl.ds(h * H_DIM, 16)] * w)

        @pl.when(ci >= 1)
        def _():
            scat_drain0(ci - 1)
        didx_wait0(ci)
        pltpu.async_copy(pay.at[b], sh.at[didx0.at[b]], ssem, add=True)

        @pl.when(ci + 1 < _SNT)
        def _():
            didx_load0(ci + 1)

        @pl.when(ci + 2 < _SNT)
        def _():
            sp_load0(ci + 2)

    @pl.when(cid == 0)
    def _():
        sp_load0(0)
        sp_load0(1)
        didx_load0(0)
        sp_wait0(0)
        vgat_issue0(0)

        def body0(i2, carry):
            for j in range(2):
                substep0(i2 * 2 + j, j)
            return carry
        lax.fori_loop(0, _SNT // 2, body0, 0)
        scat_drain0(_SNT - 1)

    def mload1(ci):
        b = ci % 2
        pltpu.async_copy(dstE.at[pl.ds(ebase(ci), _SC4)], didx1.at[b], isem)
        pltpu.async_copy(m2.at[pl.ds(ebase(ci), _SC4), :], vbuf.at[b], isem)

    def mwait1(ci):
        b = ci % 2
        pltpu.make_async_copy(dstE.at[pl.ds(e0, _SC4)], didx1.at[b],
                              isem).wait()
        pltpu.make_async_copy(m2.at[pl.ds(e0, _SC4), :], vbuf.at[b],
                              isem).wait()

    def scat_drain1(ci):
        b = ci % 2
        pltpu.make_async_copy(vbuf.at[b], sh.at[didx1.at[b]], ssem).wait()

    def substep1(ci, b):
        mwait1(ci)

        @pl.when(ci >= 1)
        def _():
            scat_drain1(ci - 1)
        pltpu.async_copy(vbuf.at[b], sh.at[didx1.at[b]], ssem, add=True)

        @pl.when(ci + 1 < _SNT)
        def _():
            mload1(ci + 1)

    @pl.when(cid == 1)
    def _():
        mload1(0)

        def body1(i2, carry):
            for j in range(2):
                substep1(i2 * 2 + j, j)
            return carry
        lax.fori_loop(0, _SNT // 2, body1, 0)
        scat_drain1(_SNT - 1)

    plsc.subcore_barrier()

    @pl.when(cid == 0)
    def _():
        pltpu.sync_copy(sh.at[pl.ds(row0, _NROW), :],
                        out0.at[pl.ds(row0, _NROW), :])

    @pl.when(cid == 1)
    def _():
        pltpu.sync_copy(sh.at[pl.ds(row0, _NROW), :],
                        out1.at[pl.ds(row0, _NROW), :])


def _scatter_stage(dst, src, pflat, m2, V, zin):
    mesh = plsc.VectorSubcoreMesh(core_axis_name="c", subcore_axis_name="s")
    f = pl.kernel(
        _p4_body,
        out_type=[jax.ShapeDtypeStruct((_NPAD, HD), jnp.float32),
                  jax.ShapeDtypeStruct((_NPAD, HD), jnp.float32)],
        mesh=mesh,
        compiler_params=pltpu.CompilerParams(needs_layout_passes=False),
        scratch_types=[
            pltpu.VMEM_SHARED((_NPAD, HD), jnp.float32),
            pltpu.VMEM((2, _SC4), jnp.int32),
            pltpu.VMEM((2, _SC4), jnp.int32),
            pltpu.VMEM((2 * _SC4,), jnp.int32),
            pltpu.VMEM((2 * _SC4 * HEADS,), jnp.float32),
            pltpu.VMEM((2, _SC4, HD), jnp.float32),
            pltpu.VMEM((2, _SC4, HD), jnp.float32),
            pltpu.SemaphoreType.DMA,
            pltpu.SemaphoreType.DMA,
            pltpu.SemaphoreType.DMA,
        ],
    )
    return f(dst, src, pflat, m2, V, zin)


_GC = 64
_GPW = E_EDGES // 32
_GN = _GPW // _GC
_GTAIL = _GPW - _GN * _GC


def _p2_body(srcE, dstE, kN, qN, gE,
             idxs, idxd, bufk, bufq, bufg, ti, tk, tq, isem, gsem, wsem):
    cid = lax.axis_index("c")
    sid = lax.axis_index("s")
    wid = sid * 2 + cid
    w0 = wid * _GPW

    def ebase(ci):
        return w0 + ci * _GC

    def idx_load(ci, b):
        pltpu.async_copy(srcE.at[pl.ds(ebase(ci), _GC)], idxs.at[b], isem)
        pltpu.async_copy(dstE.at[pl.ds(ebase(ci), _GC)], idxd.at[b], isem)

    def idx_wait(b):
        pltpu.make_async_copy(srcE.at[pl.ds(w0, _GC)], idxs.at[b], isem).wait()
        pltpu.make_async_copy(dstE.at[pl.ds(w0, _GC)], idxd.at[b], isem).wait()

    def gat_issue(b):
        pltpu.async_copy(kN.at[idxs.at[b]], bufk.at[b], gsem)
        pltpu.async_copy(qN.at[idxd.at[b]], bufq.at[b], gsem)

    def gat_wait(b):
        pltpu.make_async_copy(kN.at[idxs.at[b]], bufk.at[b], gsem).wait()
        pltpu.make_async_copy(qN.at[idxd.at[b]], bufq.at[b], gsem).wait()

    def wrt_drain(b):
        pltpu.make_async_copy(bufg.at[b], gE.at[pl.ds(w0, _GC), :],
                              wsem).wait()

    idx_load(0, 0)
    idx_wait(0)
    gat_issue(0)
    idx_load(1, 1)

    def body(i2, carry):
        for b in range(2):
            ci = i2 * 2 + b

            @pl.when(ci + 1 < _GN)
            def _():
                idx_wait(b ^ 1)
                gat_issue(b ^ 1)
            gat_wait(b)

            @pl.when(ci + 2 < _GN)
            def _():
                idx_load(ci + 2, b)

            @pl.when(ci >= 2)
            def _():
                wrt_drain(b)
            for r in range(_GC):
                for h in range(HEADS):
                    sl = pl.ds(h * H_DIM, 16)
                    bufg[b, r, sl] = bufk[b, r, sl] + bufq[b, r, sl]
            pltpu.async_copy(bufg.at[b], gE.at[pl.ds(ebase(ci), _GC), :],
                             wsem)
        return carry
    lax.fori_loop(0, _GN // 2, body, 0)
    wrt_drain(0)
    wrt_drain(1)

    tb = w0 + _GN * _GC
    pltpu.sync_copy(srcE.at[pl.ds(tb, _GTAIL)], ti)
    pltpu.async_copy(kN.at[ti], tk, gsem).wait()
    pltpu.sync_copy(dstE.at[pl.ds(tb, _GTAIL)], ti)
    pltpu.async_copy(qN.at[ti], tq, gsem).wait()
    for r in range(_GTAIL):
        for h in range(HEADS):
            sl = pl.ds(h * H_DIM, 16)
            tk[r, sl] = tk[r, sl] + tq[r, sl]
    pltpu.sync_copy(tk, gE.at[pl.ds(tb, _GTAIL), :])


def _gather_stage(src, dst, K, Q):
    mesh = plsc.VectorSubcoreMesh(core_axis_name="c", subcore_axis_name="s")
    f = pl.kernel(
        _p2_body,
        out_type=[jax.ShapeDtypeStruct((E_EDGES, HD), jnp.float32)],
        mesh=mesh,
        compiler_params=pltpu.CompilerParams(needs_layout_passes=False),
        scratch_types=[
            pltpu.VMEM((2, _GC), jnp.int32),
            pltpu.VMEM((2, _GC), jnp.int32),
            pltpu.VMEM((2, _GC, HD), jnp.float32),
            pltpu.VMEM((2, _GC, HD), jnp.float32),
            pltpu.VMEM((2, _GC, HD), jnp.float32),
            pltpu.VMEM((_GTAIL,), jnp.int32),
            pltpu.VMEM((_GTAIL, HD), jnp.float32),
            pltpu.VMEM((_GTAIL, HD), jnp.float32),
            pltpu.SemaphoreType.DMA,
            pltpu.SemaphoreType.DMA,
            pltpu.SemaphoreType.DMA,
        ],
    )
    return f(src, dst, K, Q)[0]


def _p4b_body(dstE, pflat, outS0, outS1, sloc, idxb, pbuf):
    cid = lax.axis_index("c")
    sid = lax.axis_index("s")
    wid = sid * 2 + cid
    iota16 = lax.iota(jnp.int32, 16)
    splat = [(iota16 * 0 + k).reshape(16, 1) for k in range(16)]
    msk8 = iota16 < 8
    ioff = lax.bitwise_and(iota16, 7)
    zf = iota16.astype(jnp.float32) * 0.0

    def zbody(i, carry):
        for j in range(8):
            sloc[pl.ds(i * 128 + j * 16, 16)] = zf
        return carry
    lax.fori_loop(0, _SROW * HD // 128, zbody, 0)

    nw = _NCHUNK // 32
    rem = _NCHUNK - nw * 32
    nloc = nw + jnp.where(wid < rem, 1, 0)

    def body(i, carry):
        base = (i * 32 + wid) * _C
        pltpu.sync_copy(dstE.at[pl.ds(base, _C)], idxb.at[0])
        pltpu.sync_copy(pflat.at[pl.ds(base * HEADS, _C * HEADS)], pbuf)
        for q2 in range(_C // 2):
            pp = pbuf[pl.ds(q2 * 16, 16)]
            if q2 % 8 == 0:
                dwin = idxb[0, pl.ds(q2 * 2, 16)]
            d0 = _bcast16(dwin, splat[(2 * q2) % 16])
            d1 = _bcast16(dwin, splat[(2 * q2 + 1) % 16])
            plsc.addupdate_scatter(sloc, [d0 * HEADS + ioff], pp, mask=msk8)
            plsc.addupdate_scatter(sloc, [d1 * HEADS + ioff], pp, mask=~msk8)
        return carry
    lax.fori_loop(0, nloc, body, 0)

    @pl.when(cid == 0)
    def _():
        pltpu.sync_copy(sloc, outS0.at[sid])

    @pl.when(cid == 1)
    def _():
        pltpu.sync_copy(sloc, outS1.at[sid])


def _s_stage(dst, pflat):
    mesh = plsc.VectorSubcoreMesh(core_axis_name="c", subcore_axis_name="s")
    f = pl.kernel(
        _p4b_body,
        out_type=[jax.ShapeDtypeStruct((_NSUB, _SROW * HD), jnp.float32)] * 2,
        mesh=mesh,
        compiler_params=pltpu.CompilerParams(needs_layout_passes=False),
        scratch_types=[
            pltpu.VMEM((_SROW * HD,), jnp.float32),
            pltpu.VMEM((1, _C), jnp.int32),
            pltpu.VMEM((_C * HEADS,), jnp.float32),
        ],
    )
    return f(dst, pflat)


def _sum32_body(a_ref, b_ref, out_ref):
    out_ref[...] = jnp.sum(a_ref[...], axis=0) + jnp.sum(b_ref[...], axis=0)


def _sum32(t0, t1):
    blk = _SROW * HD // 8
    return pl.pallas_call(
        _sum32_body,
        grid=(8,),
        in_specs=[pl.BlockSpec((_NSUB, blk), lambda i: (0, i))] * 2,
        out_specs=pl.BlockSpec((blk,), lambda i: (i,)),
        out_shape=jax.ShapeDtypeStruct((_SROW * HD,), jnp.float32),
    )(t0, t1)


def _node_body(q_ref, s_ref, av_ref, ac_ref, r8, veb, out_ref):
    inv = 1.0 / (s_ref[...] + 1e-30)
    inv_rep = jnp.dot(inv, r8[...], preferred_element_type=jnp.float32)
    rv = ac_ref[...] * inv_rep
    out_ref[...] = (q_ref[...] + av_ref[...] * inv_rep
                    + jnp.dot(rv, veb[...], preferred_element_type=jnp.float32))


def _node_stage(Q, S, accV, accC, R8, VeBlock):
    full = lambda s: pl.BlockSpec(s, lambda i: (0,) * len(s))
    nspec = pl.BlockSpec((_NB, HD), lambda i: (i, 0))
    return pl.pallas_call(
        _node_body,
        grid=(N // _NB,),
        in_specs=[nspec, pl.BlockSpec((_NB, HEADS), lambda i: (i, 0)),
                  nspec, nspec, full((HEADS, HD)), full((HD, HD))],
        out_specs=nspec,
        out_shape=jax.ShapeDtypeStruct((N, HD), jnp.float32),
    )(Q, S, accV, accC, R8, VeBlock)


def kernel(x, edge_index, edge_attr, WQ, bQ, WK, bK, WE, bE, WV, bV, Aw, VeRow):
    src = edge_index[0]
    dst = edge_index[1]

    h = np.arange(HEADS)
    j = np.arange(H_DIM)
    perm1 = (32 * h[:, None] + j[None, :]).reshape(-1)
    perm2 = (32 * h[:, None] + 16 + j[None, :]).reshape(-1)
    WE1, bE1 = WE[:, perm1], bE[perm1]
    WE2, bE2 = WE[:, perm2], bE[perm2]

    rows = jnp.arange(HD)
    hcol = jnp.repeat(jnp.arange(HEADS), H_DIM)
    AwBlock = jnp.zeros((HD, HEADS), jnp.float32).at[rows, hcol].set(
        Aw[:, :, 0].T.reshape(HD))
    R8 = jnp.zeros((HEADS, HD), jnp.float32).at[hcol, rows].set(1.0)

    h_i = jnp.repeat(jnp.arange(HEADS), H_DIM * H_DIM)
    d_i = jnp.tile(jnp.repeat(jnp.arange(H_DIM), H_DIM), HEADS)
    c_i = jnp.tile(jnp.arange(H_DIM), HEADS * H_DIM)
    VeBlock = jnp.zeros((HD, HD), jnp.float32).at[
        16 * h_i + d_i, 16 * h_i + c_i].set(VeRow[d_i, h_i, c_i])

    K, Q, V = _proj(x, WK, bK, WQ, bQ, WV, bV)

    G = _gather_stage(src, dst, K, Q)

    e_out, p, m2 = _edge_stage(edge_attr, G, WE1, bE1, WE2, bE2, AwBlock, R8)

    zin = jnp.zeros((_NPAD, HD), jnp.float32)
    pflat = p.reshape(-1)
    out0, out1 = _scatter_stage(dst, src, pflat, m2, V, zin)
    outS0, outS1 = _s_stage(dst, pflat)
    accV = out0[:N]
    accC = out1[:N]
    S = _sum32(outS0, outS1).reshape(_NPAD, HEADS)[:N]

    n_out = _node_stage(Q, S, accV, accC, R8, VeBlock)
    return (n_out, e_out)


# speedup vs baseline: 61.1197x; 1.0015x over previous
import functools

import jax
import jax.numpy as jnp
import numpy as np
from jax import lax
from jax.experimental import pallas as pl
from jax.experimental.pallas import tpu as pltpu
from jax.experimental.pallas import tpu_sc as plsc

N = 10000
E_EDGES = 320000
IN_DIM = 128
H_DIM = 16
HEADS = 8
HD = H_DIM * HEADS
CLAMP = 5.0

_NB = 1000
_EB = 2000


def _proj_body(x_ref, wk, bk, wq, bq, wv, bv, k_out, q_out, v_out):
    xb = x_ref[...]
    k_out[...] = jnp.dot(xb, wk[...], preferred_element_type=jnp.float32) + bk[...]
    q_out[...] = jnp.dot(xb, wq[...], preferred_element_type=jnp.float32) + bq[...]
    v_out[...] = jnp.dot(xb, wv[...], preferred_element_type=jnp.float32) + bv[...]


def _proj(x, WK, bK, WQ, bQ, WV, bV):
    full = lambda s: pl.BlockSpec(s, lambda i: (0,) * len(s))
    nspec = pl.BlockSpec((_NB, HD), lambda i: (i, 0))
    return pl.pallas_call(
        _proj_body,
        grid=(N // _NB,),
        in_specs=[pl.BlockSpec((_NB, IN_DIM), lambda i: (i, 0)),
                  full((IN_DIM, HD)), full((1, HD)),
                  full((IN_DIM, HD)), full((1, HD)),
                  full((IN_DIM, HD)), full((1, HD))],
        out_specs=[nspec, nspec, nspec],
        out_shape=[jax.ShapeDtypeStruct((N, HD), jnp.float32)] * 3,
    )(x, WK, bK.reshape(1, HD), WQ, bQ.reshape(1, HD), WV, bV.reshape(1, HD))


def _edge_body(ea_ref, g_ref, we1, be1, we2, be2, awb, r8,
               eout_ref, p_ref, m2_ref):
    ea = ea_ref[...]
    ex1 = jnp.dot(ea, we1[...], preferred_element_type=jnp.float32) + be1[...]
    ex2 = jnp.dot(ea, we2[...], preferred_element_type=jnp.float32) + be2[...]
    s2 = ex1 * ex2
    score2 = jnp.sqrt(jax.nn.relu(s2)) - jnp.sqrt(jax.nn.relu(-s2))
    conn = g_ref[...] + score2
    eout_ref[...] = conn
    score = jnp.dot(conn, awb[...], preferred_element_type=jnp.float32)
    p = jnp.exp(jnp.clip(score, -CLAMP, CLAMP))
    p_ref[...] = p
    prep = jnp.dot(p, r8[...], preferred_element_type=jnp.float32)
    m2_ref[...] = prep * conn


def _edge_stage(edge_attr, G, WE1, bE1, WE2, bE2, AwBlock, R8):
    full = lambda s: pl.BlockSpec(s, lambda i: (0,) * len(s))
    espec = pl.BlockSpec((_EB, HD), lambda i: (i, 0))
    return pl.pallas_call(
        _edge_body,
        grid=(E_EDGES // _EB,),
        in_specs=[espec, espec,
                  full((IN_DIM, HD)), full((1, HD)),
                  full((IN_DIM, HD)), full((1, HD)),
                  full((IN_DIM, HEADS)), full((HEADS, HD))],
        out_specs=[espec, pl.BlockSpec((_EB, HEADS), lambda i: (i, 0)), espec],
        out_shape=[jax.ShapeDtypeStruct((E_EDGES, HD), jnp.float32),
                   jax.ShapeDtypeStruct((E_EDGES, HEADS), jnp.float32),
                   jax.ShapeDtypeStruct((E_EDGES, HD), jnp.float32)],
    )(edge_attr, G, WE1, bE1.reshape(1, HD), WE2, bE2.reshape(1, HD),
      AwBlock, R8)


_C = 128
_NCHUNK = E_EDGES // _C
_NSUB = 16
_NPAD = 10240
_NROW = _NPAD // _NSUB
_SROW = _NPAD // 16
_GDN = lax.GatherDimensionNumbers(
    offset_dims=(), collapsed_slice_dims=(0,), start_index_map=(0,))


def _bcast16(vec, idxvec):
    return lax.gather(vec, idxvec, _GDN, (1,),
                      mode=lax.GatherScatterMode.PROMISE_IN_BOUNDS)


_SCW = E_EDGES // _NSUB
_SC4 = 80
_SNT = _SCW // _SC4


def _p4_body(dstE, srcE, pflat, m2, vN, zin, out0, out1,
             sh, didx0, didx1, sidx, pbuf, vbuf, pay, isem, gsem, ssem):
    cid = lax.axis_index("c")
    sid = lax.axis_index("s")
    iota16 = lax.iota(jnp.int32, 16)
    splat = [(iota16 * 0 + k).reshape(16, 1) for k in range(16)]
    row0 = sid * _NROW
    pltpu.sync_copy(zin.at[pl.ds(row0, _NROW), :], sh.at[pl.ds(row0, _NROW), :])
    plsc.subcore_barrier()

    e0 = sid * _SCW

    def ebase(ci):
        return e0 + ci * _SC4

    def sp_load0(ci):
        b = ci % 2
        pltpu.async_copy(srcE.at[pl.ds(ebase(ci), _SC4)],
                         sidx.at[pl.ds(b * _SC4, _SC4)], isem)
        pltpu.async_copy(pflat.at[pl.ds(ebase(ci) * HEADS, _SC4 * HEADS)],
                         pbuf.at[pl.ds(b * _SC4 * HEADS, _SC4 * HEADS)], isem)

    def sp_wait0(ci):
        b = ci % 2
        pltpu.make_async_copy(srcE.at[pl.ds(e0, _SC4)],
                              sidx.at[pl.ds(b * _SC4, _SC4)], isem).wait()
        pltpu.make_async_copy(pflat.at[pl.ds(e0, _SC4 * HEADS)],
                              pbuf.at[pl.ds(b * _SC4 * HEADS, _SC4 * HEADS)],
                              isem).wait()

    def vgat_issue0(ci):
        b = ci % 2
        pltpu.async_copy(vN.at[sidx.at[pl.ds(b * _SC4, _SC4)]], vbuf.at[b],
                         gsem)

    def vgat_wait0(ci):
        b = ci % 2
        pltpu.make_async_copy(vN.at[sidx.at[pl.ds(b * _SC4, _SC4)]],
                              vbuf.at[b], gsem).wait()

    def didx_load0(ci):
        pltpu.async_copy(dstE.at[pl.ds(ebase(ci), _SC4)], didx0.at[ci % 2],
                         isem)

    def didx_wait0(ci):
        pltpu.make_async_copy(dstE.at[pl.ds(e0, _SC4)], didx0.at[ci % 2],
                              isem).wait()

    def scat_drain0(ci):
        b = ci % 2
        pltpu.make_async_copy(pay.at[b], sh.at[didx0.at[b]], ssem).wait()

    def substep0(ci, b):
        @pl.when(ci + 1 < _SNT)
        def _():
            sp_wait0(ci + 1)
            vgat_issue0(ci + 1)
        vgat_wait0(ci)
        for q2 in range(_SC4 // 2):
            pp = pbuf[pl.ds(b * _SC4 * HEADS + q2 * 16, 16)]
            for a in range(2):
                r = 2 * q2 + a
                for h in range(HEADS):
                    w = _bcast16(pp, splat[a * HEADS + h])
                    pay[b, r, pl.ds(h * H_DIM, 16)] = (
                        vbuf[b, r, pl.ds(h * H_DIM, 16)] * w)

        @pl.when(ci >= 1)
        def _():
            scat_drain0(ci - 1)
        didx_wait0(ci)
        pltpu.async_copy(pay.at[b], sh.at[didx0.at[b]], ssem, add=True)

        @pl.when(ci + 1 < _SNT)
        def _():
            didx_load0(ci + 1)

        @pl.when(ci + 2 < _SNT)
        def _():
            sp_load0(ci + 2)

    @pl.when(cid == 0)
    def _():
        sp_load0(0)
        sp_load0(1)
        didx_load0(0)
        sp_wait0(0)
        vgat_issue0(0)

        def body0(i2, carry):
            for j in range(2):
                substep0(i2 * 2 + j, j)
            return carry
        lax.fori_loop(0, _SNT // 2, body0, 0)
        scat_drain0(_SNT - 1)

    def mload1(ci):
        b = ci % 2
        pltpu.async_copy(dstE.at[pl.ds(ebase(ci), _SC4)], didx1.at[b], isem)
        pltpu.async_copy(m2.at[pl.ds(ebase(ci), _SC4), :], vbuf.at[b], isem)

    def mwait1(ci):
        b = ci % 2
        pltpu.make_async_copy(dstE.at[pl.ds(e0, _SC4)], didx1.at[b],
                              isem).wait()
        pltpu.make_async_copy(m2.at[pl.ds(e0, _SC4), :], vbuf.at[b],
                              isem).wait()

    def scat_drain1(ci):
        b = ci % 2
        pltpu.make_async_copy(vbuf.at[b], sh.at[didx1.at[b]], ssem).wait()

    def substep1(ci, b):
        mwait1(ci)

        @pl.when(ci >= 1)
        def _():
            scat_drain1(ci - 1)
        pltpu.async_copy(vbuf.at[b], sh.at[didx1.at[b]], ssem, add=True)

        @pl.when(ci + 1 < _SNT)
        def _():
            mload1(ci + 1)

    @pl.when(cid == 1)
    def _():
        mload1(0)

        def body1(i2, carry):
            for j in range(2):
                substep1(i2 * 2 + j, j)
            return carry
        lax.fori_loop(0, _SNT // 2, body1, 0)
        scat_drain1(_SNT - 1)

    plsc.subcore_barrier()

    @pl.when(cid == 0)
    def _():
        pltpu.sync_copy(sh.at[pl.ds(row0, _NROW), :],
                        out0.at[pl.ds(row0, _NROW), :])

    @pl.when(cid == 1)
    def _():
        pltpu.sync_copy(sh.at[pl.ds(row0, _NROW), :],
                        out1.at[pl.ds(row0, _NROW), :])


def _scatter_stage(dst, src, pflat, m2, V, zin):
    mesh = plsc.VectorSubcoreMesh(core_axis_name="c", subcore_axis_name="s")
    f = pl.kernel(
        _p4_body,
        out_type=[jax.ShapeDtypeStruct((_NPAD, HD), jnp.float32),
                  jax.ShapeDtypeStruct((_NPAD, HD), jnp.float32)],
        mesh=mesh,
        compiler_params=pltpu.CompilerParams(needs_layout_passes=False),
        scratch_types=[
            pltpu.VMEM_SHARED((_NPAD, HD), jnp.float32),
            pltpu.VMEM((2, _SC4), jnp.int32),
            pltpu.VMEM((2, _SC4), jnp.int32),
            pltpu.VMEM((2 * _SC4,), jnp.int32),
            pltpu.VMEM((2 * _SC4 * HEADS,), jnp.float32),
            pltpu.VMEM((2, _SC4, HD), jnp.float32),
            pltpu.VMEM((2, _SC4, HD), jnp.float32),
            pltpu.SemaphoreType.DMA,
            pltpu.SemaphoreType.DMA,
            pltpu.SemaphoreType.DMA,
        ],
    )
    return f(dst, src, pflat, m2, V, zin)


_GC = 80
_GPW = E_EDGES // 32
_GN = _GPW // _GC


def _p2_body(srcE, dstE, kN, qN, gE,
             idxs, idxd, bufk, bufq, bufg, isem, gsem, wsem):
    cid = lax.axis_index("c")
    sid = lax.axis_index("s")
    wid = sid * 2 + cid
    w0 = wid * _GPW

    def ebase(ci):
        return w0 + ci * _GC

    def idx_load(ci, b):
        pltpu.async_copy(srcE.at[pl.ds(ebase(ci), _GC)], idxs.at[b], isem)
        pltpu.async_copy(dstE.at[pl.ds(ebase(ci), _GC)], idxd.at[b], isem)

    def idx_wait(b):
        pltpu.make_async_copy(srcE.at[pl.ds(w0, _GC)], idxs.at[b], isem).wait()
        pltpu.make_async_copy(dstE.at[pl.ds(w0, _GC)], idxd.at[b], isem).wait()

    def gat_issue(b):
        pltpu.async_copy(kN.at[idxs.at[b]], bufk.at[b], gsem)
        pltpu.async_copy(qN.at[idxd.at[b]], bufq.at[b], gsem)

    def gat_wait(b):
        pltpu.make_async_copy(kN.at[idxs.at[b]], bufk.at[b], gsem).wait()
        pltpu.make_async_copy(qN.at[idxd.at[b]], bufq.at[b], gsem).wait()

    def wrt_drain(b):
        pltpu.make_async_copy(bufg.at[b], gE.at[pl.ds(w0, _GC), :],
                              wsem).wait()

    idx_load(0, 0)
    idx_wait(0)
    gat_issue(0)
    idx_load(1, 1)

    def body(i2, carry):
        for b in range(2):
            ci = i2 * 2 + b

            @pl.when(ci + 1 < _GN)
            def _():
                idx_wait(b ^ 1)
                gat_issue(b ^ 1)
            gat_wait(b)

            @pl.when(ci + 2 < _GN)
            def _():
                idx_load(ci + 2, b)

            @pl.when(ci >= 2)
            def _():
                wrt_drain(b)
            for r in range(_GC):
                for h in range(HEADS):
                    sl = pl.ds(h * H_DIM, 16)
                    bufg[b, r, sl] = bufk[b, r, sl] + bufq[b, r, sl]
            pltpu.async_copy(bufg.at[b], gE.at[pl.ds(ebase(ci), _GC), :],
                             wsem)
        return carry
    lax.fori_loop(0, _GN // 2, body, 0)
    def estep(ci, b):
        @pl.when(ci + 1 < _GN)
        def _():
            idx_wait(b ^ 1)
            gat_issue(b ^ 1)
        gat_wait(b)
        wrt_drain(b)
        for r in range(_GC):
            for h in range(HEADS):
                sl = pl.ds(h * H_DIM, 16)
                bufg[b, r, sl] = bufk[b, r, sl] + bufq[b, r, sl]
        pltpu.async_copy(bufg.at[b], gE.at[pl.ds(ebase(ci), _GC), :], wsem)
    estep(_GN - 1, (_GN - 1) % 2)
    wrt_drain(0)
    wrt_drain(1)


def _gather_stage(src, dst, K, Q):
    mesh = plsc.VectorSubcoreMesh(core_axis_name="c", subcore_axis_name="s")
    f = pl.kernel(
        _p2_body,
        out_type=[jax.ShapeDtypeStruct((E_EDGES, HD), jnp.float32)],
        mesh=mesh,
        compiler_params=pltpu.CompilerParams(needs_layout_passes=False),
        scratch_types=[
            pltpu.VMEM((2, _GC), jnp.int32),
            pltpu.VMEM((2, _GC), jnp.int32),
            pltpu.VMEM((2, _GC, HD), jnp.float32),
            pltpu.VMEM((2, _GC, HD), jnp.float32),
            pltpu.VMEM((2, _GC, HD), jnp.float32),
            pltpu.SemaphoreType.DMA,
            pltpu.SemaphoreType.DMA,
            pltpu.SemaphoreType.DMA,
        ],
    )
    return f(src, dst, K, Q)[0]


def _p4b_body(dstE, pflat, outS0, outS1, sloc, idxb, pbuf):
    cid = lax.axis_index("c")
    sid = lax.axis_index("s")
    wid = sid * 2 + cid
    iota16 = lax.iota(jnp.int32, 16)
    splat = [(iota16 * 0 + k).reshape(16, 1) for k in range(16)]
    msk8 = iota16 < 8
    ioff = lax.bitwise_and(iota16, 7)
    zf = iota16.astype(jnp.float32) * 0.0

    def zbody(i, carry):
        for j in range(8):
            sloc[pl.ds(i * 128 + j * 16, 16)] = zf
        return carry
    lax.fori_loop(0, _SROW * HD // 128, zbody, 0)

    nw = _NCHUNK // 32
    rem = _NCHUNK - nw * 32
    nloc = nw + jnp.where(wid < rem, 1, 0)

    def body(i, carry):
        base = (i * 32 + wid) * _C
        pltpu.sync_copy(dstE.at[pl.ds(base, _C)], idxb.at[0])
        pltpu.sync_copy(pflat.at[pl.ds(base * HEADS, _C * HEADS)], pbuf)
        for q2 in range(_C // 2):
            pp = pbuf[pl.ds(q2 * 16, 16)]
            if q2 % 8 == 0:
                dwin = idxb[0, pl.ds(q2 * 2, 16)]
            d0 = _bcast16(dwin, splat[(2 * q2) % 16])
            d1 = _bcast16(dwin, splat[(2 * q2 + 1) % 16])
            plsc.addupdate_scatter(sloc, [d0 * HEADS + ioff], pp, mask=msk8)
            plsc.addupdate_scatter(sloc, [d1 * HEADS + ioff], pp, mask=~msk8)
        return carry
    lax.fori_loop(0, nloc, body, 0)

    @pl.when(cid == 0)
    def _():
        pltpu.sync_copy(sloc, outS0.at[sid])

    @pl.when(cid == 1)
    def _():
        pltpu.sync_copy(sloc, outS1.at[sid])


def _s_stage(dst, pflat):
    mesh = plsc.VectorSubcoreMesh(core_axis_name="c", subcore_axis_name="s")
    f = pl.kernel(
        _p4b_body,
        out_type=[jax.ShapeDtypeStruct((_NSUB, _SROW * HD), jnp.float32)] * 2,
        mesh=mesh,
        compiler_params=pltpu.CompilerParams(needs_layout_passes=False),
        scratch_types=[
            pltpu.VMEM((_SROW * HD,), jnp.float32),
            pltpu.VMEM((1, _C), jnp.int32),
            pltpu.VMEM((_C * HEADS,), jnp.float32),
        ],
    )
    return f(dst, pflat)


def _sum32_body(a_ref, b_ref, out_ref):
    out_ref[...] = jnp.sum(a_ref[...], axis=0) + jnp.sum(b_ref[...], axis=0)


def _sum32(t0, t1):
    blk = _SROW * HD // 8
    return pl.pallas_call(
        _sum32_body,
        grid=(8,),
        in_specs=[pl.BlockSpec((_NSUB, blk), lambda i: (0, i))] * 2,
        out_specs=pl.BlockSpec((blk,), lambda i: (i,)),
        out_shape=jax.ShapeDtypeStruct((_SROW * HD,), jnp.float32),
    )(t0, t1)


def _node_body(q_ref, s_ref, av_ref, ac_ref, r8, veb, out_ref):
    inv = 1.0 / (s_ref[...] + 1e-30)
    inv_rep = jnp.dot(inv, r8[...], preferred_element_type=jnp.float32)
    rv = ac_ref[...] * inv_rep
    out_ref[...] = (q_ref[...] + av_ref[...] * inv_rep
                    + jnp.dot(rv, veb[...], preferred_element_type=jnp.float32))


def _node_stage(Q, S, accV, accC, R8, VeBlock):
    full = lambda s: pl.BlockSpec(s, lambda i: (0,) * len(s))
    nspec = pl.BlockSpec((_NB, HD), lambda i: (i, 0))
    return pl.pallas_call(
        _node_body,
        grid=(N // _NB,),
        in_specs=[nspec, pl.BlockSpec((_NB, HEADS), lambda i: (i, 0)),
                  nspec, nspec, full((HEADS, HD)), full((HD, HD))],
        out_specs=nspec,
        out_shape=jax.ShapeDtypeStruct((N, HD), jnp.float32),
    )(Q, S, accV, accC, R8, VeBlock)


def kernel(x, edge_index, edge_attr, WQ, bQ, WK, bK, WE, bE, WV, bV, Aw, VeRow):
    src = edge_index[0]
    dst = edge_index[1]

    h = np.arange(HEADS)
    j = np.arange(H_DIM)
    perm1 = (32 * h[:, None] + j[None, :]).reshape(-1)
    perm2 = (32 * h[:, None] + 16 + j[None, :]).reshape(-1)
    WE1, bE1 = WE[:, perm1], bE[perm1]
    WE2, bE2 = WE[:, perm2], bE[perm2]

    rows = jnp.arange(HD)
    hcol = jnp.repeat(jnp.arange(HEADS), H_DIM)
    AwBlock = jnp.zeros((HD, HEADS), jnp.float32).at[rows, hcol].set(
        Aw[:, :, 0].T.reshape(HD))
    R8 = jnp.zeros((HEADS, HD), jnp.float32).at[hcol, rows].set(1.0)

    h_i = jnp.repeat(jnp.arange(HEADS), H_DIM * H_DIM)
    d_i = jnp.tile(jnp.repeat(jnp.arange(H_DIM), H_DIM), HEADS)
    c_i = jnp.tile(jnp.arange(H_DIM), HEADS * H_DIM)
    VeBlock = jnp.zeros((HD, HD), jnp.float32).at[
        16 * h_i + d_i, 16 * h_i + c_i].set(VeRow[d_i, h_i, c_i])

    K, Q, V = _proj(x, WK, bK, WQ, bQ, WV, bV)

    G = _gather_stage(src, dst, K, Q)

    e_out, p, m2 = _edge_stage(edge_attr, G, WE1, bE1, WE2, bE2, AwBlock, R8)

    zin = jnp.zeros((_NPAD, HD), jnp.float32)
    pflat = p.reshape(-1)
    out0, out1 = _scatter_stage(dst, src, pflat, m2, V, zin)
    outS0, outS1 = _s_stage(dst, pflat)
    accV = out0[:N]
    accC = out1[:N]
    S = _sum32(outS0, outS1).reshape(_NPAD, HEADS)[:N]

    n_out = _node_stage(Q, S, accV, accC, R8, VeBlock)
    return (n_out, e_out)
